# baseline (speedup 1.0000x reference)
import functools

import jax
import jax.numpy as jnp
from jax import lax
from jax.experimental import pallas as pl
from jax.experimental.pallas import tpu as pltpu

ACT_DIM = 2048
DICT_SIZE = 16384
K = 64
B = 8192


def _encode_body(x_ref, w_ref, b_dec_ref, enc_b_ref, out_ref):
    a = x_ref[...] - b_dec_ref[...]
    acc = lax.dot_general(a, w_ref[...], (((1,), (1,)), ((), ())),
                          preferred_element_type=jnp.float32)
    out_ref[...] = jnp.maximum(acc + enc_b_ref[...], 0.0)


def _encode(x, enc_W, enc_b, b_dec, m_blk=1024, f_blk=1024):
    grid = (B // m_blk, DICT_SIZE // f_blk)
    return pl.pallas_call(
        _encode_body,
        grid=grid,
        in_specs=[
            pl.BlockSpec((m_blk, ACT_DIM), lambda i, j: (i, 0)),
            pl.BlockSpec((f_blk, ACT_DIM), lambda i, j: (j, 0)),
            pl.BlockSpec((1, ACT_DIM), lambda i, j: (0, 0)),
            pl.BlockSpec((1, f_blk), lambda i, j: (0, j)),
        ],
        out_specs=pl.BlockSpec((m_blk, f_blk), lambda i, j: (i, j)),
        out_shape=jax.ShapeDtypeStruct((B, DICT_SIZE), jnp.float32),
    )(x, enc_W, b_dec.reshape(1, ACT_DIM), enc_b.reshape(1, DICT_SIZE))


def _decode_body(e_ref, w_ref, b_ref, out_ref):
    k = pl.program_id(1)

    @pl.when(k == 0)
    def _init():
        out_ref[...] = jnp.broadcast_to(b_ref[...], out_ref.shape)

    out_ref[...] += lax.dot_general(
        e_ref[...], w_ref[...], (((1,), (1,)), ((), ())),
        preferred_element_type=jnp.float32)


def _decode(encoded, dec_W, b_dec, m_blk=1024, k_blk=1024):
    grid = (B // m_blk, DICT_SIZE // k_blk)
    return pl.pallas_call(
        _decode_body,
        grid=grid,
        in_specs=[
            pl.BlockSpec((m_blk, k_blk), lambda i, k: (i, k)),
            pl.BlockSpec((ACT_DIM, k_blk), lambda i, k: (0, k)),
            pl.BlockSpec((1, ACT_DIM), lambda i, k: (0, 0)),
        ],
        out_specs=pl.BlockSpec((m_blk, ACT_DIM), lambda i, k: (i, 0)),
        out_shape=jax.ShapeDtypeStruct((B, ACT_DIM), jnp.float32),
    )(encoded, dec_W, b_dec.reshape(1, ACT_DIM))


def kernel(x, enc_W, enc_b, dec_W, b_dec):
    post = _encode(x, enc_W, enc_b, b_dec)
    vals, idx = jax.lax.top_k(post, K)
    row_idx = jnp.arange(B)[:, None]
    encoded = jnp.zeros_like(post).at[row_idx, idx].set(vals)
    x_hat = _decode(encoded, dec_W, b_dec)
    return x_hat


# speedup vs baseline: 3.9808x; 3.9808x over previous
import functools

import jax
import jax.numpy as jnp
from jax import lax
from jax.experimental import pallas as pl
from jax.experimental.pallas import tpu as pltpu

from jax.experimental.pallas import tpu_sc as plsc

ACT_DIM = 2048
DICT_SIZE = 16384
K = 64
B = 8192

NC = 2
NS = 16
NW = NC * NS
ROWS_PER_W = B // NW
NV = DICT_SIZE // 16


def _sc_topk_mask(post_flat):
    def _suffix_scan(hist_ref, sufx_ref, nch):
        def body(jj, carry):
            j = nch - 1 - jj
            v = hist_ref[pl.ds(j * 16, 16)]
            rv = lax.rev(v, (0,))
            c = jnp.cumsum(rv) + carry
            sufx_ref[pl.ds(j * 16, 16)] = lax.rev(c, (0,))
            return jnp.max(c)
        lax.fori_loop(0, nch, body, jnp.int32(0))

    def _find_bucket(sufx_ref, nch, rem):
        def body(j, cnt):
            s = sufx_ref[pl.ds(j * 16, 16)]
            return cnt + jnp.sum((s >= rem).astype(jnp.int32))
        cnt = lax.fori_loop(0, nch, body, jnp.int32(0))
        bstar = cnt - 1
        nbkt = nch * 16
        nxt = jnp.minimum(bstar + 1, nbkt - 1)
        sv = plsc.load_gather(sufx_ref, [jnp.full((16,), nxt, jnp.int32)])
        n_gt = jnp.where(bstar + 1 >= nbkt, 0, jnp.max(sv))
        mv = plsc.load_gather(sufx_ref, [jnp.full((16,), bstar, jnp.int32)])
        return bstar, n_gt, jnp.max(mv)

    def body(post_hbm, out_hbm, row_v, out_v, cand_v, hist_v, sufx_v):
        iota16 = jnp.arange(16, dtype=jnp.int32)
        ones16 = jnp.ones((16,), jnp.int32)
        zeros16 = jnp.zeros((16,), jnp.int32)
        wid = lax.axis_index("s") * NC + lax.axis_index("c")

        def row_body(r, _carry):
            base = (wid * ROWS_PER_W + r) * DICT_SIZE
            pltpu.sync_copy(post_hbm.at[pl.ds(base, DICT_SIZE)], row_v)

            def zh(j, _):
                hist_v[pl.ds(j * 16, 16)] = zeros16
                return 0
            lax.fori_loop(0, 32, zh, 0)

            def h1(jo, _):
                for u in range(8):
                    j = jo * 8 + u
                    v = row_v[pl.ds(j * 16, 16)]
                    b = jnp.right_shift(lax.bitcast_convert_type(v, jnp.int32), 22)
                    plsc.addupdate_scatter(hist_v, [b], ones16)
                return 0
            lax.fori_loop(0, NV // 8, h1, 0)

            _suffix_scan(hist_v, sufx_v, 32)
            b1, n_gt, m = _find_bucket(sufx_v, 32, jnp.int32(K))
            rem = K - n_gt

            def cp(jo, off):
                for u in range(4):
                    j = jo * 4 + u
                    v = row_v[pl.ds(j * 16, 16)]
                    b = jnp.right_shift(lax.bitcast_convert_type(v, jnp.int32), 22)
                    mask = b >= b1
                    mi = mask.astype(jnp.int32)
                    pos = jnp.maximum(off + jnp.cumsum(mi) - 1, 0)
                    plsc.store_scatter(cand_v, [pos], v, mask=mask)
                    off = off + jnp.sum(mi)
                return off
            lax.fori_loop(0, NV // 4, cp, jnp.int32(0))

            pv = b1
            for lo_prev, lo, fm, nch in ((22, 13, 0x1FF, 32),
                                         (13, 4, 0x1FF, 32),
                                         (4, 0, 0xF, 1)):
                lax.fori_loop(0, nch, zh, 0)
                nloops = (m + 15) // 16

                def hl(j, _, lo_prev=lo_prev, lo=lo, fm=fm, pv=pv):
                    v = cand_v[pl.ds(j * 16, 16)]
                    bits = lax.bitcast_convert_type(v, jnp.int32)
                    valid = (j * 16 + iota16) < m
                    act = valid & (jnp.right_shift(bits, lo_prev) == pv)
                    field = jnp.right_shift(bits, lo) & fm
                    plsc.addupdate_scatter(hist_v, [field], ones16,
                                           mask=act)
                    return 0
                lax.fori_loop(0, nloops, hl, 0)

                _suffix_scan(hist_v, sufx_v, nch)
                bl, n_gt, _m2 = _find_bucket(sufx_v, nch, rem)
                pv = jnp.left_shift(pv, lo_prev - lo) | bl
                rem = rem - n_gt

            tvec = lax.bitcast_convert_type(jnp.full((16,), pv, jnp.int32), jnp.float32)

            def fin(jo, eqc):
                for u in range(4):
                    j = jo * 4 + u
                    v = row_v[pl.ds(j * 16, 16)]
                    gt = v > tvec
                    eq = v == tvec
                    ei = eq.astype(jnp.int32)
                    rank = eqc + jnp.cumsum(ei) - 1
                    keep = gt | (eq & (rank < rem))
                    out_v[pl.ds(j * 16, 16)] = jnp.where(keep, v, 0.0)
                    eqc = eqc + jnp.sum(ei)
                return eqc
            lax.fori_loop(0, NV // 4, fin, jnp.int32(0))

            pltpu.sync_copy(out_v, out_hbm.at[pl.ds(base, DICT_SIZE)])
            return 0

        lax.fori_loop(0, ROWS_PER_W, row_body, 0)

    mesh = plsc.VectorSubcoreMesh(core_axis_name="c", subcore_axis_name="s")
    f = pl.kernel(
        body,
        mesh=mesh,
        compiler_params=pltpu.CompilerParams(needs_layout_passes=False),
        out_type=jax.ShapeDtypeStruct((B * DICT_SIZE,), jnp.float32),
        scratch_types=[
            pltpu.VMEM((DICT_SIZE,), jnp.float32),
            pltpu.VMEM((DICT_SIZE,), jnp.float32),
            pltpu.VMEM((DICT_SIZE,), jnp.float32),
            pltpu.VMEM((512,), jnp.int32),
            pltpu.VMEM((512,), jnp.int32),
        ],
    )
    return f(post_flat)


def _encode_body(x_ref, w_ref, b_dec_ref, enc_b_ref, out_ref):
    a = x_ref[...] - b_dec_ref[...]
    acc = lax.dot_general(a, w_ref[...], (((1,), (1,)), ((), ())),
                          preferred_element_type=jnp.float32)
    out_ref[...] = jnp.maximum(acc + enc_b_ref[...], 0.0)


def _encode(x, enc_W, enc_b, b_dec, m_blk=1024, f_blk=1024):
    grid = (B // m_blk, DICT_SIZE // f_blk)
    return pl.pallas_call(
        _encode_body,
        grid=grid,
        in_specs=[
            pl.BlockSpec((m_blk, ACT_DIM), lambda i, j: (i, 0)),
            pl.BlockSpec((f_blk, ACT_DIM), lambda i, j: (j, 0)),
            pl.BlockSpec((1, ACT_DIM), lambda i, j: (0, 0)),
            pl.BlockSpec((1, f_blk), lambda i, j: (0, j)),
        ],
        out_specs=pl.BlockSpec((m_blk, f_blk), lambda i, j: (i, j)),
        out_shape=jax.ShapeDtypeStruct((B, DICT_SIZE), jnp.float32),
    )(x, enc_W, b_dec.reshape(1, ACT_DIM), enc_b.reshape(1, DICT_SIZE))


def _decode_body(e_ref, w_ref, b_ref, out_ref):
    k = pl.program_id(1)

    @pl.when(k == 0)
    def _init():
        out_ref[...] = jnp.broadcast_to(b_ref[...], out_ref.shape)

    out_ref[...] += lax.dot_general(
        e_ref[...], w_ref[...], (((1,), (1,)), ((), ())),
        preferred_element_type=jnp.float32)


def _decode(encoded, dec_W, b_dec, m_blk=1024, k_blk=1024):
    grid = (B // m_blk, DICT_SIZE // k_blk)
    return pl.pallas_call(
        _decode_body,
        grid=grid,
        in_specs=[
            pl.BlockSpec((m_blk, k_blk), lambda i, k: (i, k)),
            pl.BlockSpec((ACT_DIM, k_blk), lambda i, k: (0, k)),
            pl.BlockSpec((1, ACT_DIM), lambda i, k: (0, 0)),
        ],
        out_specs=pl.BlockSpec((m_blk, ACT_DIM), lambda i, k: (i, 0)),
        out_shape=jax.ShapeDtypeStruct((B, ACT_DIM), jnp.float32),
    )(encoded, dec_W, b_dec.reshape(1, ACT_DIM))


def kernel(x, enc_W, enc_b, dec_W, b_dec):
    post = _encode(x, enc_W, enc_b, b_dec)
    encoded = _sc_topk_mask(post.reshape(-1)).reshape(B, DICT_SIZE)
    x_hat = _decode(encoded, dec_W, b_dec)
    return x_hat


# speedup vs baseline: 4.0964x; 1.0290x over previous
import functools

import jax
import jax.numpy as jnp
from jax import lax
from jax.experimental import pallas as pl
from jax.experimental.pallas import tpu as pltpu

from jax.experimental.pallas import tpu_sc as plsc

ACT_DIM = 2048
DICT_SIZE = 16384
K = 64
B = 8192

NC = 2
NS = 16
NW = NC * NS
ROWS_PER_W = B // NW
NV = DICT_SIZE // 16


def _lane(vec, lane):
    return lax.gather(
        vec, jnp.full((16, 1), lane, jnp.int32),
        lax.GatherDimensionNumbers(offset_dims=(), collapsed_slice_dims=(0,),
                                   start_index_map=(0,)),
        (1,), mode=lax.GatherScatterMode.PROMISE_IN_BOUNDS)


def _popcnt(mask):
    return plsc.all_reduce_population_count(mask)


def _sc_topk_mask(post_flat):
    def _suffix_scan(hist_ref, sufx_ref, nch):
        def body(jj, carry):
            j = nch - 1 - jj
            v = hist_ref[pl.ds(j * 16, 16)]
            rv = lax.rev(v, (0,))
            c = jnp.cumsum(rv) + carry
            sufx_ref[pl.ds(j * 16, 16)] = lax.rev(c, (0,))
            return _lane(c, 15)
        lax.fori_loop(0, nch, body, jnp.zeros((16,), jnp.int32))

    def _find_bucket(sufx_ref, nch, rem):
        def body(j, cnt):
            s = sufx_ref[pl.ds(j * 16, 16)]
            return cnt + _popcnt(s >= rem)
        cnt = lax.fori_loop(0, nch, body, jnp.zeros((16,), jnp.int32))
        bstar = cnt - 1
        nbkt = nch * 16
        nxt = jnp.minimum(bstar + 1, nbkt - 1)
        sv = plsc.load_gather(sufx_ref, [nxt])
        n_gt = jnp.where(bstar + 1 >= nbkt, 0, sv)
        m = plsc.load_gather(sufx_ref, [jnp.maximum(bstar, 0)])
        return bstar, n_gt, m

    def body(post_hbm, out_hbm, row_v, out_v, cand_v, hist_v, sufx_v):
        iota16 = jnp.arange(16, dtype=jnp.int32)
        ones16 = jnp.ones((16,), jnp.int32)
        zeros16 = jnp.zeros((16,), jnp.int32)
        wid = lax.axis_index("s") * NC + lax.axis_index("c")

        def row_body(r, _carry):
            base = (wid * ROWS_PER_W + r) * DICT_SIZE
            pltpu.sync_copy(post_hbm.at[pl.ds(base, DICT_SIZE)], row_v)

            def zh(j, _):
                hist_v[pl.ds(j * 16, 16)] = zeros16
                return 0
            lax.fori_loop(0, 32, zh, 0)

            def h1(jo, _):
                for u in range(8):
                    j = jo * 8 + u
                    v = row_v[pl.ds(j * 16, 16)]
                    b = jnp.right_shift(lax.bitcast_convert_type(v, jnp.int32), 22)
                    plsc.addupdate_scatter(hist_v, [b], ones16)
                return 0
            lax.fori_loop(0, NV // 8, h1, 0)

            _suffix_scan(hist_v, sufx_v, 32)
            b1, n_gt, m = _find_bucket(sufx_v, 32, jnp.full((16,), K, jnp.int32))
            rem = K - n_gt

            def cp(jo, off):
                for u in range(4):
                    j = jo * 4 + u
                    v = row_v[pl.ds(j * 16, 16)]
                    b = jnp.right_shift(lax.bitcast_convert_type(v, jnp.int32), 22)
                    mask = b >= b1
                    mi = mask.astype(jnp.int32)
                    pos = jnp.maximum(off + jnp.cumsum(mi) - 1, 0)
                    plsc.store_scatter(cand_v, [pos], v, mask=mask)
                    off = off + _popcnt(mask)
                return off
            lax.fori_loop(0, NV // 4, cp, jnp.zeros((16,), jnp.int32))

            m_s = jnp.max(m)
            nloops = (m_s + 15) // 16

            pv = b1
            for lo_prev, lo, fm, nch in ((22, 14, 0xFF, 16),
                                         (14, 6, 0xFF, 16),
                                         (6, 0, 0x3F, 4)):
                lax.fori_loop(0, nch, zh, 0)

                def hl(j, _, lo_prev=lo_prev, lo=lo, fm=fm, pv=pv):
                    v = cand_v[pl.ds(j * 16, 16)]
                    bits = lax.bitcast_convert_type(v, jnp.int32)
                    valid = (j * 16 + iota16) < m
                    act = valid & (jnp.right_shift(bits, lo_prev) == pv)
                    field = jnp.right_shift(bits, lo) & fm
                    plsc.addupdate_scatter(hist_v, [field], ones16,
                                           mask=act)
                    return 0
                lax.fori_loop(0, nloops, hl, 0)

                _suffix_scan(hist_v, sufx_v, nch)
                bl, n_gt, _m2 = _find_bucket(sufx_v, nch, rem)
                pv = jnp.left_shift(pv, lo_prev - lo) | bl
                rem = rem - n_gt

            tvec = lax.bitcast_convert_type(pv, jnp.float32)

            def fin(jo, eqc):
                for u in range(4):
                    j = jo * 4 + u
                    v = row_v[pl.ds(j * 16, 16)]
                    gt = v > tvec
                    eq = v == tvec
                    ei = eq.astype(jnp.int32)
                    rank = eqc + jnp.cumsum(ei) - 1
                    keep = gt | (eq & (rank < rem))
                    out_v[pl.ds(j * 16, 16)] = jnp.where(keep, v, 0.0)
                    eqc = eqc + _popcnt(eq)
                return eqc
            lax.fori_loop(0, NV // 4, fin, jnp.zeros((16,), jnp.int32))

            pltpu.sync_copy(out_v, out_hbm.at[pl.ds(base, DICT_SIZE)])
            return 0

        lax.fori_loop(0, ROWS_PER_W, row_body, 0)

    mesh = plsc.VectorSubcoreMesh(core_axis_name="c", subcore_axis_name="s")
    f = pl.kernel(
        body,
        mesh=mesh,
        compiler_params=pltpu.CompilerParams(needs_layout_passes=False),
        out_type=jax.ShapeDtypeStruct((B * DICT_SIZE,), jnp.float32),
        scratch_types=[
            pltpu.VMEM((DICT_SIZE,), jnp.float32),
            pltpu.VMEM((DICT_SIZE,), jnp.float32),
            pltpu.VMEM((DICT_SIZE,), jnp.float32),
            pltpu.VMEM((512,), jnp.int32),
            pltpu.VMEM((512,), jnp.int32),
        ],
    )
    return f(post_flat)


def _encode_body(x_ref, w_ref, b_dec_ref, enc_b_ref, out_ref):
    a = x_ref[...] - b_dec_ref[...]
    acc = lax.dot_general(a, w_ref[...], (((1,), (1,)), ((), ())),
                          preferred_element_type=jnp.float32)
    out_ref[...] = jnp.maximum(acc + enc_b_ref[...], 0.0)


def _encode(x, enc_W, enc_b, b_dec, m_blk=1024, f_blk=1024):
    grid = (B // m_blk, DICT_SIZE // f_blk)
    return pl.pallas_call(
        _encode_body,
        grid=grid,
        in_specs=[
            pl.BlockSpec((m_blk, ACT_DIM), lambda i, j: (i, 0)),
            pl.BlockSpec((f_blk, ACT_DIM), lambda i, j: (j, 0)),
            pl.BlockSpec((1, ACT_DIM), lambda i, j: (0, 0)),
            pl.BlockSpec((1, f_blk), lambda i, j: (0, j)),
        ],
        out_specs=pl.BlockSpec((m_blk, f_blk), lambda i, j: (i, j)),
        out_shape=jax.ShapeDtypeStruct((B, DICT_SIZE), jnp.float32),
    )(x, enc_W, b_dec.reshape(1, ACT_DIM), enc_b.reshape(1, DICT_SIZE))


def _decode_body(e_ref, w_ref, b_ref, out_ref):
    k = pl.program_id(1)

    @pl.when(k == 0)
    def _init():
        out_ref[...] = jnp.broadcast_to(b_ref[...], out_ref.shape)

    out_ref[...] += lax.dot_general(
        e_ref[...], w_ref[...], (((1,), (1,)), ((), ())),
        preferred_element_type=jnp.float32)


def _decode(encoded, dec_W, b_dec, m_blk=1024, k_blk=1024):
    grid = (B // m_blk, DICT_SIZE // k_blk)
    return pl.pallas_call(
        _decode_body,
        grid=grid,
        in_specs=[
            pl.BlockSpec((m_blk, k_blk), lambda i, k: (i, k)),
            pl.BlockSpec((ACT_DIM, k_blk), lambda i, k: (0, k)),
            pl.BlockSpec((1, ACT_DIM), lambda i, k: (0, 0)),
        ],
        out_specs=pl.BlockSpec((m_blk, ACT_DIM), lambda i, k: (i, 0)),
        out_shape=jax.ShapeDtypeStruct((B, ACT_DIM), jnp.float32),
    )(encoded, dec_W, b_dec.reshape(1, ACT_DIM))


def kernel(x, enc_W, enc_b, dec_W, b_dec):
    post = _encode(x, enc_W, enc_b, b_dec)
    encoded = _sc_topk_mask(post.reshape(-1)).reshape(B, DICT_SIZE)
    x_hat = _decode(encoded, dec_W, b_dec)
    return x_hat


# speedup vs baseline: 7.3594x; 1.7966x over previous
import functools

import jax
import jax.numpy as jnp
from jax import lax
from jax.experimental import pallas as pl
from jax.experimental.pallas import tpu as pltpu

from jax.experimental.pallas import tpu_sc as plsc

ACT_DIM = 2048
DICT_SIZE = 16384
K = 64
B = 8192

NC = 2
NS = 16
NW = NC * NS
ROWS_PER_W = B // NW
NV = DICT_SIZE // 16


def _lane(vec, lane):
    return lax.gather(
        vec, jnp.full((16, 1), lane, jnp.int32),
        lax.GatherDimensionNumbers(offset_dims=(), collapsed_slice_dims=(0,),
                                   start_index_map=(0,)),
        (1,), mode=lax.GatherScatterMode.PROMISE_IN_BOUNDS)


def _popcnt(mask):
    return plsc.all_reduce_population_count(mask)


def _sc_topk_mask(post_flat):
    def _suffix_scan(hist_ref, sufx_ref, nch):
        @plsc.parallel_loop(0, nch, unroll=4,
                            carry=jnp.zeros((16,), jnp.int32))
        def _(jj, carry):
            j = nch - 1 - jj
            v = hist_ref[pl.ds(j * 16, 16)]
            rv = lax.rev(v, (0,))
            c = jnp.cumsum(rv) + carry
            sufx_ref[pl.ds(j * 16, 16)] = lax.rev(c, (0,))
            return _lane(c, 15)

    def _find_bucket(sufx_ref, nch, rem):
        @plsc.parallel_loop(0, nch, unroll=4,
                            carry=jnp.zeros((16,), jnp.int32))
        def cnt(j, cnt):
            s = sufx_ref[pl.ds(j * 16, 16)]
            return cnt + _popcnt(s >= rem)
        bstar = cnt - 1
        nbkt = nch * 16
        nxt = jnp.minimum(bstar + 1, nbkt - 1)
        sv = plsc.load_gather(sufx_ref, [nxt])
        n_gt = jnp.where(bstar + 1 >= nbkt, 0, sv)
        m = plsc.load_gather(sufx_ref, [jnp.maximum(bstar, 0)])
        return bstar, n_gt, m

    def body(post_hbm, out_hbm, row_v, out_v, cand_v, hist_v, sufx_v):
        iota16 = jnp.arange(16, dtype=jnp.int32)
        ones16 = jnp.ones((16,), jnp.int32)
        zeros16 = jnp.zeros((16,), jnp.int32)
        wid = lax.axis_index("s") * NC + lax.axis_index("c")

        def row_body(r, _carry):
            base = (wid * ROWS_PER_W + r) * DICT_SIZE
            pltpu.sync_copy(post_hbm.at[pl.ds(base, DICT_SIZE)], row_v)

            def zero_hist(nch):
                @plsc.parallel_loop(0, nch, unroll=4)
                def _(j):
                    hist_v[pl.ds(j * 16, 16)] = zeros16
            zero_hist(32)

            @plsc.parallel_loop(0, NV, unroll=8)
            def _(j):
                v = row_v[pl.ds(j * 16, 16)]
                b = jnp.right_shift(lax.bitcast_convert_type(v, jnp.int32), 22)
                plsc.addupdate_scatter(hist_v, [b], ones16)

            _suffix_scan(hist_v, sufx_v, 32)
            b1, n_gt, m = _find_bucket(sufx_v, 32, jnp.full((16,), K, jnp.int32))
            rem = K - n_gt

            @plsc.parallel_loop(0, NV, unroll=8,
                                carry=jnp.zeros((16,), jnp.int32))
            def _(j, off):
                v = row_v[pl.ds(j * 16, 16)]
                b = jnp.right_shift(lax.bitcast_convert_type(v, jnp.int32), 22)
                mask = b >= b1
                mi = mask.astype(jnp.int32)
                pos = jnp.maximum(off + jnp.cumsum(mi) - 1, 0)
                plsc.store_scatter(cand_v, [pos], v, mask=mask)
                return off + _popcnt(mask)

            m_s = jnp.max(m)
            nloops = (m_s + 15) // 16

            pv = b1
            for lo_prev, lo, fm, nch in ((22, 14, 0xFF, 16),
                                         (14, 6, 0xFF, 16),
                                         (6, 0, 0x3F, 4)):
                zero_hist(nch)

                def hl(j, _, lo_prev=lo_prev, lo=lo, fm=fm, pv=pv):
                    v = cand_v[pl.ds(j * 16, 16)]
                    bits = lax.bitcast_convert_type(v, jnp.int32)
                    valid = (j * 16 + iota16) < m
                    act = valid & (jnp.right_shift(bits, lo_prev) == pv)
                    field = jnp.right_shift(bits, lo) & fm
                    plsc.addupdate_scatter(hist_v, [field], ones16,
                                           mask=act)
                    return 0
                lax.fori_loop(0, nloops, hl, 0)

                _suffix_scan(hist_v, sufx_v, nch)
                bl, n_gt, _m2 = _find_bucket(sufx_v, nch, rem)
                pv = jnp.left_shift(pv, lo_prev - lo) | bl
                rem = rem - n_gt

            tvec = lax.bitcast_convert_type(pv, jnp.float32)

            @plsc.parallel_loop(0, NV, unroll=8,
                                carry=jnp.zeros((16,), jnp.int32))
            def _(j, eqc):
                v = row_v[pl.ds(j * 16, 16)]
                gt = v > tvec
                eq = v == tvec
                ei = eq.astype(jnp.int32)
                rank = eqc + jnp.cumsum(ei) - 1
                keep = gt | (eq & (rank < rem))
                out_v[pl.ds(j * 16, 16)] = jnp.where(keep, v, 0.0)
                return eqc + _popcnt(eq)

            pltpu.sync_copy(out_v, out_hbm.at[pl.ds(base, DICT_SIZE)])
            return 0

        lax.fori_loop(0, ROWS_PER_W, row_body, 0)

    mesh = plsc.VectorSubcoreMesh(core_axis_name="c", subcore_axis_name="s")
    f = pl.kernel(
        body,
        mesh=mesh,
        compiler_params=pltpu.CompilerParams(needs_layout_passes=False),
        out_type=jax.ShapeDtypeStruct((B * DICT_SIZE,), jnp.float32),
        scratch_types=[
            pltpu.VMEM((DICT_SIZE,), jnp.float32),
            pltpu.VMEM((DICT_SIZE,), jnp.float32),
            pltpu.VMEM((DICT_SIZE,), jnp.float32),
            pltpu.VMEM((512,), jnp.int32),
            pltpu.VMEM((512,), jnp.int32),
        ],
    )
    return f(post_flat)


def _encode_body(x_ref, w_ref, b_dec_ref, enc_b_ref, out_ref):
    a = x_ref[...] - b_dec_ref[...]
    acc = lax.dot_general(a, w_ref[...], (((1,), (1,)), ((), ())),
                          preferred_element_type=jnp.float32)
    out_ref[...] = jnp.maximum(acc + enc_b_ref[...], 0.0)


def _encode(x, enc_W, enc_b, b_dec, m_blk=1024, f_blk=1024):
    grid = (B // m_blk, DICT_SIZE // f_blk)
    return pl.pallas_call(
        _encode_body,
        grid=grid,
        in_specs=[
            pl.BlockSpec((m_blk, ACT_DIM), lambda i, j: (i, 0)),
            pl.BlockSpec((f_blk, ACT_DIM), lambda i, j: (j, 0)),
            pl.BlockSpec((1, ACT_DIM), lambda i, j: (0, 0)),
            pl.BlockSpec((1, f_blk), lambda i, j: (0, j)),
        ],
        out_specs=pl.BlockSpec((m_blk, f_blk), lambda i, j: (i, j)),
        out_shape=jax.ShapeDtypeStruct((B, DICT_SIZE), jnp.float32),
    )(x, enc_W, b_dec.reshape(1, ACT_DIM), enc_b.reshape(1, DICT_SIZE))


def _decode_body(e_ref, w_ref, b_ref, out_ref):
    k = pl.program_id(1)

    @pl.when(k == 0)
    def _init():
        out_ref[...] = jnp.broadcast_to(b_ref[...], out_ref.shape)

    out_ref[...] += lax.dot_general(
        e_ref[...], w_ref[...], (((1,), (1,)), ((), ())),
        preferred_element_type=jnp.float32)


def _decode(encoded, dec_W, b_dec, m_blk=1024, k_blk=1024):
    grid = (B // m_blk, DICT_SIZE // k_blk)
    return pl.pallas_call(
        _decode_body,
        grid=grid,
        in_specs=[
            pl.BlockSpec((m_blk, k_blk), lambda i, k: (i, k)),
            pl.BlockSpec((ACT_DIM, k_blk), lambda i, k: (0, k)),
            pl.BlockSpec((1, ACT_DIM), lambda i, k: (0, 0)),
        ],
        out_specs=pl.BlockSpec((m_blk, ACT_DIM), lambda i, k: (i, 0)),
        out_shape=jax.ShapeDtypeStruct((B, ACT_DIM), jnp.float32),
    )(encoded, dec_W, b_dec.reshape(1, ACT_DIM))


def kernel(x, enc_W, enc_b, dec_W, b_dec):
    post = _encode(x, enc_W, enc_b, b_dec)
    encoded = _sc_topk_mask(post.reshape(-1)).reshape(B, DICT_SIZE)
    x_hat = _decode(encoded, dec_W, b_dec)
    return x_hat


# speedup vs baseline: 7.9539x; 1.0808x over previous
import functools

import jax
import jax.numpy as jnp
from jax import lax
from jax.experimental import pallas as pl
from jax.experimental.pallas import tpu as pltpu

from jax.experimental.pallas import tpu_sc as plsc

ACT_DIM = 2048
DICT_SIZE = 16384
K = 64
B = 8192

NC = 2
NS = 16
NW = NC * NS
ROWS_PER_W = B // NW
NV = DICT_SIZE // 16


def _lane(vec, lane):
    return lax.gather(
        vec, jnp.full((16, 1), lane, jnp.int32),
        lax.GatherDimensionNumbers(offset_dims=(), collapsed_slice_dims=(0,),
                                   start_index_map=(0,)),
        (1,), mode=lax.GatherScatterMode.PROMISE_IN_BOUNDS)


def _popcnt(mask):
    return plsc.all_reduce_population_count(mask)


def _sc_topk_mask(post_flat):
    def _suffix_scan(hist_ref, sufx_ref, nch):
        @plsc.parallel_loop(0, nch, unroll=4,
                            carry=jnp.zeros((16,), jnp.int32))
        def _(jj, carry):
            j = nch - 1 - jj
            v = hist_ref[pl.ds(j * 16, 16)]
            rv = lax.rev(v, (0,))
            c = jnp.cumsum(rv) + carry
            sufx_ref[pl.ds(j * 16, 16)] = lax.rev(c, (0,))
            return _lane(c, 15)

    def _find_bucket(sufx_ref, nch, rem):
        @plsc.parallel_loop(0, nch, unroll=4,
                            carry=jnp.zeros((16,), jnp.int32))
        def cnt(j, cnt):
            s = sufx_ref[pl.ds(j * 16, 16)]
            return cnt + _popcnt(s >= rem)
        bstar = cnt - 1
        nbkt = nch * 16
        nxt = jnp.minimum(bstar + 1, nbkt - 1)
        sv = plsc.load_gather(sufx_ref, [nxt])
        n_gt = jnp.where(bstar + 1 >= nbkt, 0, sv)
        m = plsc.load_gather(sufx_ref, [jnp.maximum(bstar, 0)])
        return bstar, n_gt, m

    def body(post_hbm, out_hbm, row_a, row_b, out_a, out_b, cand_v, hist_v,
             sufx_v, sem_la, sem_lb, sem_sa, sem_sb):
        iota16 = jnp.arange(16, dtype=jnp.int32)
        ones16 = jnp.ones((16,), jnp.int32)
        zeros16 = jnp.zeros((16,), jnp.int32)
        wid = lax.axis_index("s") * NC + lax.axis_index("c")
        row0 = wid * ROWS_PER_W

        def load(buf, sem, r):
            pltpu.make_async_copy(
                post_hbm.at[pl.ds((row0 + r) * DICT_SIZE, DICT_SIZE)],
                buf, sem).start()

        def load_wait(buf, sem, r):
            pltpu.make_async_copy(
                post_hbm.at[pl.ds((row0 + r) * DICT_SIZE, DICT_SIZE)],
                buf, sem).wait()

        def store(buf, sem, r):
            pltpu.make_async_copy(
                buf, out_hbm.at[pl.ds((row0 + r) * DICT_SIZE, DICT_SIZE)],
                sem).start()

        def store_wait(buf, sem, r):
            pltpu.make_async_copy(
                buf, out_hbm.at[pl.ds((row0 + r) * DICT_SIZE, DICT_SIZE)],
                sem).wait()

        def process(row_v, out_v, wait_prev_store):

            def zero_hist(nch):
                @plsc.parallel_loop(0, nch, unroll=4)
                def _(j):
                    hist_v[pl.ds(j * 16, 16)] = zeros16
            zero_hist(32)

            @plsc.parallel_loop(0, NV, unroll=8)
            def _(j):
                v = row_v[pl.ds(j * 16, 16)]
                b = jnp.right_shift(lax.bitcast_convert_type(v, jnp.int32), 22)
                plsc.addupdate_scatter(hist_v, [b], ones16)

            _suffix_scan(hist_v, sufx_v, 32)
            b1, n_gt, m = _find_bucket(sufx_v, 32, jnp.full((16,), K, jnp.int32))
            rem = K - n_gt

            @plsc.parallel_loop(0, NV, unroll=8,
                                carry=jnp.zeros((16,), jnp.int32))
            def _(j, off):
                v = row_v[pl.ds(j * 16, 16)]
                b = jnp.right_shift(lax.bitcast_convert_type(v, jnp.int32), 22)
                mask = b >= b1
                mi = mask.astype(jnp.int32)
                pos = jnp.maximum(off + jnp.cumsum(mi) - 1, 0)
                plsc.store_scatter(cand_v, [pos], v, mask=mask)
                return off + _popcnt(mask)

            m_s = jnp.max(m)
            nloops = (m_s + 15) // 16

            pv = b1
            for lo_prev, lo, fm, nch in ((22, 14, 0xFF, 16),
                                         (14, 6, 0xFF, 16),
                                         (6, 0, 0x3F, 4)):
                zero_hist(nch)

                def hl(j, _, lo_prev=lo_prev, lo=lo, fm=fm, pv=pv):
                    v = cand_v[pl.ds(j * 16, 16)]
                    bits = lax.bitcast_convert_type(v, jnp.int32)
                    valid = (j * 16 + iota16) < m
                    act = valid & (jnp.right_shift(bits, lo_prev) == pv)
                    field = jnp.right_shift(bits, lo) & fm
                    plsc.addupdate_scatter(hist_v, [field], ones16,
                                           mask=act)
                    return 0
                lax.fori_loop(0, nloops, hl, 0)

                _suffix_scan(hist_v, sufx_v, nch)
                bl, n_gt, _m2 = _find_bucket(sufx_v, nch, rem)
                pv = jnp.left_shift(pv, lo_prev - lo) | bl
                rem = rem - n_gt

            tvec = lax.bitcast_convert_type(pv, jnp.float32)
            wait_prev_store()

            @plsc.parallel_loop(0, NV, unroll=8,
                                carry=jnp.zeros((16,), jnp.int32))
            def _(j, eqc):
                v = row_v[pl.ds(j * 16, 16)]
                gt = v > tvec
                eq = v == tvec
                ei = eq.astype(jnp.int32)
                rank = eqc + jnp.cumsum(ei) - 1
                keep = gt | (eq & (rank < rem))
                out_v[pl.ds(j * 16, 16)] = jnp.where(keep, v, 0.0)
                return eqc + _popcnt(eq)

        load(row_a, sem_la, 0)

        def pair_body(i, _):
            r = i * 2
            load(row_b, sem_lb, r + 1)
            load_wait(row_a, sem_la, r)
            process(row_a, out_a,
                    lambda: pl.when(i > 0)(lambda: store_wait(out_a, sem_sa,
                                                             r - 2)))
            store(out_a, sem_sa, r)
            @pl.when(i < ROWS_PER_W // 2 - 1)
            def _():
                load(row_a, sem_la, r + 2)
            load_wait(row_b, sem_lb, r + 1)
            process(row_b, out_b,
                    lambda: pl.when(i > 0)(lambda: store_wait(out_b, sem_sb,
                                                              r - 1)))
            store(out_b, sem_sb, r + 1)
            return 0

        lax.fori_loop(0, ROWS_PER_W // 2, pair_body, 0)
        store_wait(out_a, sem_sa, ROWS_PER_W - 2)
        store_wait(out_b, sem_sb, ROWS_PER_W - 1)

    mesh = plsc.VectorSubcoreMesh(core_axis_name="c", subcore_axis_name="s")
    f = pl.kernel(
        body,
        mesh=mesh,
        compiler_params=pltpu.CompilerParams(needs_layout_passes=False),
        out_type=jax.ShapeDtypeStruct((B * DICT_SIZE,), jnp.float32),
        scratch_types=[
            pltpu.VMEM((DICT_SIZE,), jnp.float32),
            pltpu.VMEM((DICT_SIZE,), jnp.float32),
            pltpu.VMEM((DICT_SIZE,), jnp.float32),
            pltpu.VMEM((DICT_SIZE,), jnp.float32),
            pltpu.VMEM((DICT_SIZE,), jnp.float32),
            pltpu.VMEM((512,), jnp.int32),
            pltpu.VMEM((512,), jnp.int32),
            pltpu.SemaphoreType.DMA,
            pltpu.SemaphoreType.DMA,
            pltpu.SemaphoreType.DMA,
            pltpu.SemaphoreType.DMA,
        ],
    )
    return f(post_flat)


def _encode_body(x_ref, w_ref, b_dec_ref, enc_b_ref, out_ref):
    a = x_ref[...] - b_dec_ref[...]
    acc = lax.dot_general(a, w_ref[...], (((1,), (1,)), ((), ())),
                          preferred_element_type=jnp.float32)
    out_ref[...] = jnp.maximum(acc + enc_b_ref[...], 0.0)


def _encode(x, enc_W, enc_b, b_dec, m_blk=1024, f_blk=1024):
    grid = (B // m_blk, DICT_SIZE // f_blk)
    return pl.pallas_call(
        _encode_body,
        grid=grid,
        in_specs=[
            pl.BlockSpec((m_blk, ACT_DIM), lambda i, j: (i, 0)),
            pl.BlockSpec((f_blk, ACT_DIM), lambda i, j: (j, 0)),
            pl.BlockSpec((1, ACT_DIM), lambda i, j: (0, 0)),
            pl.BlockSpec((1, f_blk), lambda i, j: (0, j)),
        ],
        out_specs=pl.BlockSpec((m_blk, f_blk), lambda i, j: (i, j)),
        out_shape=jax.ShapeDtypeStruct((B, DICT_SIZE), jnp.float32),
    )(x, enc_W, b_dec.reshape(1, ACT_DIM), enc_b.reshape(1, DICT_SIZE))


def _decode_body(e_ref, w_ref, b_ref, out_ref):
    k = pl.program_id(1)

    @pl.when(k == 0)
    def _init():
        out_ref[...] = jnp.broadcast_to(b_ref[...], out_ref.shape)

    out_ref[...] += lax.dot_general(
        e_ref[...], w_ref[...], (((1,), (1,)), ((), ())),
        preferred_element_type=jnp.float32)


def _decode(encoded, dec_W, b_dec, m_blk=1024, k_blk=1024):
    grid = (B // m_blk, DICT_SIZE // k_blk)
    return pl.pallas_call(
        _decode_body,
        grid=grid,
        in_specs=[
            pl.BlockSpec((m_blk, k_blk), lambda i, k: (i, k)),
            pl.BlockSpec((ACT_DIM, k_blk), lambda i, k: (0, k)),
            pl.BlockSpec((1, ACT_DIM), lambda i, k: (0, 0)),
        ],
        out_specs=pl.BlockSpec((m_blk, ACT_DIM), lambda i, k: (i, 0)),
        out_shape=jax.ShapeDtypeStruct((B, ACT_DIM), jnp.float32),
    )(encoded, dec_W, b_dec.reshape(1, ACT_DIM))


def kernel(x, enc_W, enc_b, dec_W, b_dec):
    post = _encode(x, enc_W, enc_b, b_dec)
    encoded = _sc_topk_mask(post.reshape(-1)).reshape(B, DICT_SIZE)
    x_hat = _decode(encoded, dec_W, b_dec)
    return x_hat


# speedup vs baseline: 8.7798x; 1.1038x over previous
import functools

import jax
import jax.numpy as jnp
from jax import lax
from jax.experimental import pallas as pl
from jax.experimental.pallas import tpu as pltpu

from jax.experimental.pallas import tpu_sc as plsc

ACT_DIM = 2048
DICT_SIZE = 16384
K = 64
B = 8192

NC = 2
NS = 16
NW = NC * NS
ROWS_PER_W = B // NW
NV = DICT_SIZE // 16


def _lane(vec, lane):
    return lax.gather(
        vec, jnp.full((16, 1), lane, jnp.int32),
        lax.GatherDimensionNumbers(offset_dims=(), collapsed_slice_dims=(0,),
                                   start_index_map=(0,)),
        (1,), mode=lax.GatherScatterMode.PROMISE_IN_BOUNDS)


def _popcnt(mask):
    return plsc.all_reduce_population_count(mask)


def _sc_topk_mask(post_flat):
    def _suffix_scan(hist_ref, sufx_ref, nch):
        @plsc.parallel_loop(0, nch, unroll=4,
                            carry=jnp.zeros((16,), jnp.int32))
        def _(jj, carry):
            j = nch - 1 - jj
            v = hist_ref[pl.ds(j * 16, 16)]
            rv = lax.rev(v, (0,))
            c = jnp.cumsum(rv) + carry
            sufx_ref[pl.ds(j * 16, 16)] = lax.rev(c, (0,))
            return _lane(c, 15)

    def _find_bucket(sufx_ref, nch, rem):
        @plsc.parallel_loop(0, nch, unroll=4,
                            carry=jnp.zeros((16,), jnp.int32))
        def cnt(j, cnt):
            s = sufx_ref[pl.ds(j * 16, 16)]
            return cnt + _popcnt(s >= rem)
        bstar = cnt - 1
        nbkt = nch * 16
        nxt = jnp.minimum(bstar + 1, nbkt - 1)
        sv = plsc.load_gather(sufx_ref, [nxt])
        n_gt = jnp.where(bstar + 1 >= nbkt, 0, sv)
        m = plsc.load_gather(sufx_ref, [jnp.maximum(bstar, 0)])
        return bstar, n_gt, m

    def body(post_hbm, out_hbm, row_a, row_b, out_a, out_b, cand_v, hist_v,
             sufx_v, sem_la, sem_lb, sem_sa, sem_sb):
        iota16 = jnp.arange(16, dtype=jnp.int32)
        ones16 = jnp.ones((16,), jnp.int32)
        zeros16 = jnp.zeros((16,), jnp.int32)
        wid = lax.axis_index("s") * NC + lax.axis_index("c")
        row0 = wid * ROWS_PER_W

        def load(buf, sem, r):
            pltpu.make_async_copy(
                post_hbm.at[pl.ds((row0 + r) * DICT_SIZE, DICT_SIZE)],
                buf, sem).start()

        def load_wait(buf, sem, r):
            pltpu.make_async_copy(
                post_hbm.at[pl.ds((row0 + r) * DICT_SIZE, DICT_SIZE)],
                buf, sem).wait()

        def store(buf, sem, r):
            pltpu.make_async_copy(
                buf, out_hbm.at[pl.ds((row0 + r) * DICT_SIZE, DICT_SIZE)],
                sem).start()

        def store_wait(buf, sem, r):
            pltpu.make_async_copy(
                buf, out_hbm.at[pl.ds((row0 + r) * DICT_SIZE, DICT_SIZE)],
                sem).wait()

        def process(row_v, out_v, wait_prev_store):

            def zero_hist(nch):
                @plsc.parallel_loop(0, nch, unroll=4)
                def _(j):
                    hist_v[pl.ds(j * 16, 16)] = zeros16
            zero_hist(32)

            @plsc.parallel_loop(0, NV, unroll=8)
            def _(j):
                v = row_v[pl.ds(j * 16, 16)]
                b = jnp.right_shift(lax.bitcast_convert_type(v, jnp.int32), 22)
                plsc.addupdate_scatter(hist_v, [b], ones16)

            _suffix_scan(hist_v, sufx_v, 32)
            b1, n_gt, m = _find_bucket(sufx_v, 32, jnp.full((16,), K, jnp.int32))
            rem = K - n_gt

            @plsc.parallel_loop(0, NV, unroll=8,
                                carry=jnp.zeros((16,), jnp.int32))
            def _(j, off):
                v = row_v[pl.ds(j * 16, 16)]
                b = jnp.right_shift(lax.bitcast_convert_type(v, jnp.int32), 22)
                mask = b >= b1
                mi = mask.astype(jnp.int32)
                pos = jnp.maximum(off + jnp.cumsum(mi) - 1, 0)
                plsc.store_scatter(cand_v, [pos], v, mask=mask)
                return off + _popcnt(mask)

            m_s = jnp.max(m)
            nloops = (m_s + 15) // 16

            pv = b1
            eq_total = m - n_gt
            for lo_prev, lo, fm, nch in ((22, 14, 0xFF, 16),
                                         (14, 6, 0xFF, 16),
                                         (6, 0, 0x3F, 4)):
                zero_hist(nch)

                def hl(j, _, lo_prev=lo_prev, lo=lo, fm=fm, pv=pv):
                    v = cand_v[pl.ds(j * 16, 16)]
                    bits = lax.bitcast_convert_type(v, jnp.int32)
                    valid = (j * 16 + iota16) < m
                    act = valid & (jnp.right_shift(bits, lo_prev) == pv)
                    field = jnp.right_shift(bits, lo) & fm
                    plsc.addupdate_scatter(hist_v, [field], ones16,
                                           mask=act)
                    return 0
                lax.fori_loop(0, nloops, hl, 0)

                _suffix_scan(hist_v, sufx_v, nch)
                bl, n_gt, m2 = _find_bucket(sufx_v, nch, rem)
                pv = jnp.left_shift(pv, lo_prev - lo) | bl
                eq_total = m2 - n_gt
                rem = rem - n_gt

            tvec = lax.bitcast_convert_type(pv, jnp.float32)
            wait_prev_store()

            def fin_fast():
                @plsc.parallel_loop(0, NV, unroll=8)
                def _(j):
                    v = row_v[pl.ds(j * 16, 16)]
                    out_v[pl.ds(j * 16, 16)] = jnp.where(v >= tvec, v, 0.0)

            def fin_slow():
                @plsc.parallel_loop(0, NV, unroll=8,
                                    carry=jnp.zeros((16,), jnp.int32))
                def _(j, eqc):
                    v = row_v[pl.ds(j * 16, 16)]
                    gt = v > tvec
                    eq = v == tvec
                    ei = eq.astype(jnp.int32)
                    rank = eqc + jnp.cumsum(ei) - 1
                    keep = gt | (eq & (rank < rem))
                    out_v[pl.ds(j * 16, 16)] = jnp.where(keep, v, 0.0)
                    return eqc + _popcnt(eq)

            lax.cond(jnp.max(eq_total) == jnp.max(rem), fin_fast, fin_slow)

        load(row_a, sem_la, 0)

        def pair_body(i, _):
            r = i * 2
            load(row_b, sem_lb, r + 1)
            load_wait(row_a, sem_la, r)
            process(row_a, out_a,
                    lambda: pl.when(i > 0)(lambda: store_wait(out_a, sem_sa,
                                                             r - 2)))
            store(out_a, sem_sa, r)
            @pl.when(i < ROWS_PER_W // 2 - 1)
            def _():
                load(row_a, sem_la, r + 2)
            load_wait(row_b, sem_lb, r + 1)
            process(row_b, out_b,
                    lambda: pl.when(i > 0)(lambda: store_wait(out_b, sem_sb,
                                                              r - 1)))
            store(out_b, sem_sb, r + 1)
            return 0

        lax.fori_loop(0, ROWS_PER_W // 2, pair_body, 0)
        store_wait(out_a, sem_sa, ROWS_PER_W - 2)
        store_wait(out_b, sem_sb, ROWS_PER_W - 1)

    mesh = plsc.VectorSubcoreMesh(core_axis_name="c", subcore_axis_name="s")
    f = pl.kernel(
        body,
        mesh=mesh,
        compiler_params=pltpu.CompilerParams(needs_layout_passes=False),
        out_type=jax.ShapeDtypeStruct((B * DICT_SIZE,), jnp.float32),
        scratch_types=[
            pltpu.VMEM((DICT_SIZE,), jnp.float32),
            pltpu.VMEM((DICT_SIZE,), jnp.float32),
            pltpu.VMEM((DICT_SIZE,), jnp.float32),
            pltpu.VMEM((DICT_SIZE,), jnp.float32),
            pltpu.VMEM((DICT_SIZE,), jnp.float32),
            pltpu.VMEM((512,), jnp.int32),
            pltpu.VMEM((512,), jnp.int32),
            pltpu.SemaphoreType.DMA,
            pltpu.SemaphoreType.DMA,
            pltpu.SemaphoreType.DMA,
            pltpu.SemaphoreType.DMA,
        ],
    )
    return f(post_flat)


def _encode_body(x_ref, w_ref, b_dec_ref, enc_b_ref, out_ref):
    a = x_ref[...] - b_dec_ref[...]
    acc = lax.dot_general(a, w_ref[...], (((1,), (1,)), ((), ())),
                          preferred_element_type=jnp.float32)
    out_ref[...] = jnp.maximum(acc + enc_b_ref[...], 0.0)


def _encode(x, enc_W, enc_b, b_dec, m_blk=1024, f_blk=1024):
    grid = (B // m_blk, DICT_SIZE // f_blk)
    return pl.pallas_call(
        _encode_body,
        grid=grid,
        in_specs=[
            pl.BlockSpec((m_blk, ACT_DIM), lambda i, j: (i, 0)),
            pl.BlockSpec((f_blk, ACT_DIM), lambda i, j: (j, 0)),
            pl.BlockSpec((1, ACT_DIM), lambda i, j: (0, 0)),
            pl.BlockSpec((1, f_blk), lambda i, j: (0, j)),
        ],
        out_specs=pl.BlockSpec((m_blk, f_blk), lambda i, j: (i, j)),
        out_shape=jax.ShapeDtypeStruct((B, DICT_SIZE), jnp.float32),
    )(x, enc_W, b_dec.reshape(1, ACT_DIM), enc_b.reshape(1, DICT_SIZE))


def _decode_body(e_ref, w_ref, b_ref, out_ref):
    k = pl.program_id(1)

    @pl.when(k == 0)
    def _init():
        out_ref[...] = jnp.broadcast_to(b_ref[...], out_ref.shape)

    out_ref[...] += lax.dot_general(
        e_ref[...], w_ref[...], (((1,), (1,)), ((), ())),
        preferred_element_type=jnp.float32)


def _decode(encoded, dec_W, b_dec, m_blk=1024, k_blk=1024):
    grid = (B // m_blk, DICT_SIZE // k_blk)
    return pl.pallas_call(
        _decode_body,
        grid=grid,
        in_specs=[
            pl.BlockSpec((m_blk, k_blk), lambda i, k: (i, k)),
            pl.BlockSpec((ACT_DIM, k_blk), lambda i, k: (0, k)),
            pl.BlockSpec((1, ACT_DIM), lambda i, k: (0, 0)),
        ],
        out_specs=pl.BlockSpec((m_blk, ACT_DIM), lambda i, k: (i, 0)),
        out_shape=jax.ShapeDtypeStruct((B, ACT_DIM), jnp.float32),
    )(encoded, dec_W, b_dec.reshape(1, ACT_DIM))


def kernel(x, enc_W, enc_b, dec_W, b_dec):
    post = _encode(x, enc_W, enc_b, b_dec)
    encoded = _sc_topk_mask(post.reshape(-1)).reshape(B, DICT_SIZE)
    x_hat = _decode(encoded, dec_W, b_dec)
    return x_hat


# speedup vs baseline: 10.6348x; 1.2113x over previous
import functools

import jax
import jax.numpy as jnp
from jax import lax
from jax.experimental import pallas as pl
from jax.experimental.pallas import tpu as pltpu

from jax.experimental.pallas import tpu_sc as plsc

ACT_DIM = 2048
DICT_SIZE = 16384
K = 64
B = 8192

NC = 2
NS = 16
NW = NC * NS
ROWS_PER_W = B // NW
NV = DICT_SIZE // 16


def _lane(vec, lane):
    return lax.gather(
        vec, jnp.full((16, 1), lane, jnp.int32),
        lax.GatherDimensionNumbers(offset_dims=(), collapsed_slice_dims=(0,),
                                   start_index_map=(0,)),
        (1,), mode=lax.GatherScatterMode.PROMISE_IN_BOUNDS)


def _popcnt(mask):
    return plsc.all_reduce_population_count(mask)


def _sc_topk_mask(post_flat, nrows=B):
    rows_per_w = nrows // NW
    def _suffix_scan(hist_ref, sufx_ref, nch):
        @plsc.parallel_loop(0, nch, unroll=4,
                            carry=jnp.zeros((16,), jnp.int32))
        def _(jj, carry):
            j = nch - 1 - jj
            v = hist_ref[pl.ds(j * 16, 16)]
            rv = lax.rev(v, (0,))
            c = jnp.cumsum(rv) + carry
            sufx_ref[pl.ds(j * 16, 16)] = lax.rev(c, (0,))
            return _lane(c, 15)

    def _find_bucket(sufx_ref, nch, rem):
        @plsc.parallel_loop(0, nch, unroll=4,
                            carry=jnp.zeros((16,), jnp.int32))
        def cnt(j, cnt):
            s = sufx_ref[pl.ds(j * 16, 16)]
            return cnt + _popcnt(s >= rem)
        bstar = cnt - 1
        nbkt = nch * 16
        nxt = jnp.minimum(bstar + 1, nbkt - 1)
        sv = plsc.load_gather(sufx_ref, [nxt])
        n_gt = jnp.where(bstar + 1 >= nbkt, 0, sv)
        m = plsc.load_gather(sufx_ref, [jnp.maximum(bstar, 0)])
        return bstar, n_gt, m

    def body(post_hbm, out_hbm, row_a, row_b, out_a, out_b, cand_v, hist_v,
             sufx_v, sem_la, sem_lb, sem_sa, sem_sb):
        iota16 = jnp.arange(16, dtype=jnp.int32)
        ones16 = jnp.ones((16,), jnp.int32)
        zeros16 = jnp.zeros((16,), jnp.int32)
        wid = lax.axis_index("s") * NC + lax.axis_index("c")
        row0 = wid * rows_per_w

        def load(buf, sem, r):
            pltpu.make_async_copy(
                post_hbm.at[pl.ds((row0 + r) * DICT_SIZE, DICT_SIZE)],
                buf, sem).start()

        def load_wait(buf, sem, r):
            pltpu.make_async_copy(
                post_hbm.at[pl.ds((row0 + r) * DICT_SIZE, DICT_SIZE)],
                buf, sem).wait()

        def store(buf, sem, r):
            pltpu.make_async_copy(
                buf, out_hbm.at[pl.ds((row0 + r) * DICT_SIZE, DICT_SIZE)],
                sem).start()

        def store_wait(buf, sem, r):
            pltpu.make_async_copy(
                buf, out_hbm.at[pl.ds((row0 + r) * DICT_SIZE, DICT_SIZE)],
                sem).wait()

        def process(row_v, out_v, wait_prev_store):

            def zero_hist(nch):
                @plsc.parallel_loop(0, nch, unroll=4)
                def _(j):
                    hist_v[pl.ds(j * 16, 16)] = zeros16
            zero_hist(32)

            @plsc.parallel_loop(0, NV, unroll=8)
            def _(j):
                v = row_v[pl.ds(j * 16, 16)]
                b = jnp.right_shift(lax.bitcast_convert_type(v, jnp.int32), 22)
                plsc.addupdate_scatter(hist_v, [b], ones16)

            _suffix_scan(hist_v, sufx_v, 32)
            b1, n_gt, m = _find_bucket(sufx_v, 32, jnp.full((16,), K, jnp.int32))
            rem = K - n_gt

            @plsc.parallel_loop(0, NV, unroll=8,
                                carry=jnp.zeros((16,), jnp.int32))
            def _(j, off):
                v = row_v[pl.ds(j * 16, 16)]
                b = jnp.right_shift(lax.bitcast_convert_type(v, jnp.int32), 22)
                mask = b >= b1
                mi = mask.astype(jnp.int32)
                pos = jnp.maximum(off + jnp.cumsum(mi) - 1, 0)
                plsc.store_scatter(cand_v, [pos], v, mask=mask)
                return off + _popcnt(mask)

            m_s = jnp.max(m)
            nloops = (m_s + 15) // 16

            pv = b1
            eq_total = m - n_gt
            for lo_prev, lo, fm, nch in ((22, 14, 0xFF, 16),
                                         (14, 6, 0xFF, 16),
                                         (6, 0, 0x3F, 4)):
                zero_hist(nch)

                def hl(j, _, lo_prev=lo_prev, lo=lo, fm=fm, pv=pv):
                    v = cand_v[pl.ds(j * 16, 16)]
                    bits = lax.bitcast_convert_type(v, jnp.int32)
                    valid = (j * 16 + iota16) < m
                    act = valid & (jnp.right_shift(bits, lo_prev) == pv)
                    field = jnp.right_shift(bits, lo) & fm
                    plsc.addupdate_scatter(hist_v, [field], ones16,
                                           mask=act)
                    return 0
                lax.fori_loop(0, nloops, hl, 0)

                _suffix_scan(hist_v, sufx_v, nch)
                bl, n_gt, m2 = _find_bucket(sufx_v, nch, rem)
                pv = jnp.left_shift(pv, lo_prev - lo) | bl
                eq_total = m2 - n_gt
                rem = rem - n_gt

            tvec = lax.bitcast_convert_type(pv, jnp.float32)
            wait_prev_store()

            def fin_fast():
                @plsc.parallel_loop(0, NV, unroll=8)
                def _(j):
                    v = row_v[pl.ds(j * 16, 16)]
                    out_v[pl.ds(j * 16, 16)] = jnp.where(v >= tvec, v, 0.0)

            def fin_slow():
                @plsc.parallel_loop(0, NV, unroll=8,
                                    carry=jnp.zeros((16,), jnp.int32))
                def _(j, eqc):
                    v = row_v[pl.ds(j * 16, 16)]
                    gt = v > tvec
                    eq = v == tvec
                    ei = eq.astype(jnp.int32)
                    rank = eqc + jnp.cumsum(ei) - 1
                    keep = gt | (eq & (rank < rem))
                    out_v[pl.ds(j * 16, 16)] = jnp.where(keep, v, 0.0)
                    return eqc + _popcnt(eq)

            lax.cond(jnp.max(eq_total) == jnp.max(rem), fin_fast, fin_slow)

        load(row_a, sem_la, 0)

        def pair_body(i, _):
            r = i * 2
            load(row_b, sem_lb, r + 1)
            load_wait(row_a, sem_la, r)
            process(row_a, out_a,
                    lambda: pl.when(i > 0)(lambda: store_wait(out_a, sem_sa,
                                                             r - 2)))
            store(out_a, sem_sa, r)
            @pl.when(i < rows_per_w // 2 - 1)
            def _():
                load(row_a, sem_la, r + 2)
            load_wait(row_b, sem_lb, r + 1)
            process(row_b, out_b,
                    lambda: pl.when(i > 0)(lambda: store_wait(out_b, sem_sb,
                                                              r - 1)))
            store(out_b, sem_sb, r + 1)
            return 0

        lax.fori_loop(0, rows_per_w // 2, pair_body, 0)
        store_wait(out_a, sem_sa, rows_per_w - 2)
        store_wait(out_b, sem_sb, rows_per_w - 1)

    mesh = plsc.VectorSubcoreMesh(core_axis_name="c", subcore_axis_name="s")
    f = pl.kernel(
        body,
        mesh=mesh,
        compiler_params=pltpu.CompilerParams(needs_layout_passes=False),
        out_type=jax.ShapeDtypeStruct((nrows * DICT_SIZE,), jnp.float32),
        scratch_types=[
            pltpu.VMEM((DICT_SIZE,), jnp.float32),
            pltpu.VMEM((DICT_SIZE,), jnp.float32),
            pltpu.VMEM((DICT_SIZE,), jnp.float32),
            pltpu.VMEM((DICT_SIZE,), jnp.float32),
            pltpu.VMEM((DICT_SIZE,), jnp.float32),
            pltpu.VMEM((512,), jnp.int32),
            pltpu.VMEM((512,), jnp.int32),
            pltpu.SemaphoreType.DMA,
            pltpu.SemaphoreType.DMA,
            pltpu.SemaphoreType.DMA,
            pltpu.SemaphoreType.DMA,
        ],
    )
    return f(post_flat)


def _encode_body(x_ref, w_ref, b_dec_ref, enc_b_ref, out_ref):
    a = x_ref[...] - b_dec_ref[...]
    acc = lax.dot_general(a, w_ref[...], (((1,), (1,)), ((), ())),
                          preferred_element_type=jnp.float32)
    out_ref[...] = jnp.maximum(acc + enc_b_ref[...], 0.0)


def _encode(x, enc_W, enc_b, b_dec, m_blk=1024, f_blk=1024):
    nb = x.shape[0]
    grid = (nb // m_blk, DICT_SIZE // f_blk)
    return pl.pallas_call(
        _encode_body,
        grid=grid,
        in_specs=[
            pl.BlockSpec((m_blk, ACT_DIM), lambda i, j: (i, 0)),
            pl.BlockSpec((f_blk, ACT_DIM), lambda i, j: (j, 0)),
            pl.BlockSpec((1, ACT_DIM), lambda i, j: (0, 0)),
            pl.BlockSpec((1, f_blk), lambda i, j: (0, j)),
        ],
        out_specs=pl.BlockSpec((m_blk, f_blk), lambda i, j: (i, j)),
        out_shape=jax.ShapeDtypeStruct((nb, DICT_SIZE), jnp.float32),
    )(x, enc_W, b_dec.reshape(1, ACT_DIM), enc_b.reshape(1, DICT_SIZE))


def _decode_body(e_ref, w_ref, b_ref, out_ref):
    k = pl.program_id(1)

    @pl.when(k == 0)
    def _init():
        out_ref[...] = jnp.broadcast_to(b_ref[...], out_ref.shape)

    out_ref[...] += lax.dot_general(
        e_ref[...], w_ref[...], (((1,), (1,)), ((), ())),
        preferred_element_type=jnp.float32)


def _decode(encoded, dec_W, b_dec, m_blk=1024, k_blk=1024):
    nb = encoded.shape[0]
    grid = (nb // m_blk, DICT_SIZE // k_blk)
    return pl.pallas_call(
        _decode_body,
        grid=grid,
        in_specs=[
            pl.BlockSpec((m_blk, k_blk), lambda i, k: (i, k)),
            pl.BlockSpec((ACT_DIM, k_blk), lambda i, k: (0, k)),
            pl.BlockSpec((1, ACT_DIM), lambda i, k: (0, 0)),
        ],
        out_specs=pl.BlockSpec((m_blk, ACT_DIM), lambda i, k: (i, 0)),
        out_shape=jax.ShapeDtypeStruct((nb, ACT_DIM), jnp.float32),
    )(encoded, dec_W, b_dec.reshape(1, ACT_DIM))


def kernel(x, enc_W, enc_b, dec_W, b_dec):
    nchunks = 4
    mb = B // nchunks
    outs = []
    for c in range(nchunks):
        xc = lax.slice_in_dim(x, c * mb, (c + 1) * mb)
        post = _encode(xc, enc_W, enc_b, b_dec)
        enc = _sc_topk_mask(post.reshape(-1), mb).reshape(mb, DICT_SIZE)
        outs.append(_decode(enc, dec_W, b_dec))
    return jnp.concatenate(outs, axis=0)


# speedup vs baseline: 10.8560x; 1.0208x over previous
import functools

import jax
import jax.numpy as jnp
from jax import lax
from jax.experimental import pallas as pl
from jax.experimental.pallas import tpu as pltpu

from jax.experimental.pallas import tpu_sc as plsc

ACT_DIM = 2048
DICT_SIZE = 16384
K = 64
B = 8192

NC = 2
NS = 16
NW = NC * NS
ROWS_PER_W = B // NW
NV = DICT_SIZE // 16


def _lane(vec, lane):
    return lax.gather(
        vec, jnp.full((16, 1), lane, jnp.int32),
        lax.GatherDimensionNumbers(offset_dims=(), collapsed_slice_dims=(0,),
                                   start_index_map=(0,)),
        (1,), mode=lax.GatherScatterMode.PROMISE_IN_BOUNDS)


def _popcnt(mask):
    return plsc.all_reduce_population_count(mask)


def _sc_topk_mask(post_flat, nrows=B):
    rows_per_w = nrows // NW
    def _suffix_scan(hist_ref, sufx_ref, nch):
        @plsc.parallel_loop(0, nch, unroll=4,
                            carry=jnp.zeros((16,), jnp.int32))
        def _(jj, carry):
            j = nch - 1 - jj
            v = hist_ref[pl.ds(j * 16, 16)]
            rv = lax.rev(v, (0,))
            c = jnp.cumsum(rv) + carry
            sufx_ref[pl.ds(j * 16, 16)] = lax.rev(c, (0,))
            return _lane(c, 15)

    def _find_bucket(sufx_ref, nch, rem):
        @plsc.parallel_loop(0, nch, unroll=4,
                            carry=jnp.zeros((16,), jnp.int32))
        def cnt(j, cnt):
            s = sufx_ref[pl.ds(j * 16, 16)]
            return cnt + _popcnt(s >= rem)
        bstar = cnt - 1
        nbkt = nch * 16
        nxt = jnp.minimum(bstar + 1, nbkt - 1)
        sv = plsc.load_gather(sufx_ref, [nxt])
        n_gt = jnp.where(bstar + 1 >= nbkt, 0, sv)
        m = plsc.load_gather(sufx_ref, [jnp.maximum(bstar, 0)])
        return bstar, n_gt, m

    def body(post_hbm, out_hbm, row_a, row_b, out_a, out_b, cand_v, hist_v,
             sufx_v, sem_la, sem_lb, sem_sa, sem_sb):
        iota16 = jnp.arange(16, dtype=jnp.int32)
        ones16 = jnp.ones((16,), jnp.int32)
        zeros16 = jnp.zeros((16,), jnp.int32)
        wid = lax.axis_index("s") * NC + lax.axis_index("c")
        row0 = wid * rows_per_w

        def load(buf, sem, r):
            pltpu.make_async_copy(
                post_hbm.at[pl.ds((row0 + r) * DICT_SIZE, DICT_SIZE)],
                buf, sem).start()

        def load_wait(buf, sem, r):
            pltpu.make_async_copy(
                post_hbm.at[pl.ds((row0 + r) * DICT_SIZE, DICT_SIZE)],
                buf, sem).wait()

        def store(buf, sem, r):
            pltpu.make_async_copy(
                buf, out_hbm.at[pl.ds((row0 + r) * DICT_SIZE, DICT_SIZE)],
                sem).start()

        def store_wait(buf, sem, r):
            pltpu.make_async_copy(
                buf, out_hbm.at[pl.ds((row0 + r) * DICT_SIZE, DICT_SIZE)],
                sem).wait()

        def process(row_v, out_v, wait_prev_store):

            def zero_hist(nch):
                @plsc.parallel_loop(0, nch, unroll=4)
                def _(j):
                    hist_v[pl.ds(j * 16, 16)] = zeros16
            zero_hist(32)

            @plsc.parallel_loop(0, NV, unroll=16)
            def _(j):
                v = row_v[pl.ds(j * 16, 16)]
                b = jnp.right_shift(lax.bitcast_convert_type(v, jnp.int32), 22)
                plsc.addupdate_scatter(hist_v, [b], ones16)

            _suffix_scan(hist_v, sufx_v, 32)
            b1, n_gt, m = _find_bucket(sufx_v, 32, jnp.full((16,), K, jnp.int32))
            rem = K - n_gt

            @plsc.parallel_loop(0, NV, unroll=8,
                                carry=jnp.zeros((16,), jnp.int32))
            def _(j, off):
                v = row_v[pl.ds(j * 16, 16)]
                b = jnp.right_shift(lax.bitcast_convert_type(v, jnp.int32), 22)
                mask = b >= b1
                mi = mask.astype(jnp.int32)
                pos = jnp.maximum(off + jnp.cumsum(mi) - 1, 0)
                plsc.store_scatter(cand_v, [pos], v, mask=mask)
                return off + _popcnt(mask)

            m_s = jnp.max(m)
            nloops = (m_s + 15) // 16

            pv = b1
            eq_total = m - n_gt
            for lo_prev, lo, fm, nch in ((22, 14, 0xFF, 16),
                                         (14, 6, 0xFF, 16),
                                         (6, 0, 0x3F, 4)):
                zero_hist(nch)

                def hl(j, _, lo_prev=lo_prev, lo=lo, fm=fm, pv=pv):
                    v = cand_v[pl.ds(j * 16, 16)]
                    bits = lax.bitcast_convert_type(v, jnp.int32)
                    valid = (j * 16 + iota16) < m
                    act = valid & (jnp.right_shift(bits, lo_prev) == pv)
                    field = jnp.right_shift(bits, lo) & fm
                    plsc.addupdate_scatter(hist_v, [field], ones16,
                                           mask=act)
                    return 0
                lax.fori_loop(0, nloops, hl, 0)

                _suffix_scan(hist_v, sufx_v, nch)
                bl, n_gt, m2 = _find_bucket(sufx_v, nch, rem)
                pv = jnp.left_shift(pv, lo_prev - lo) | bl
                eq_total = m2 - n_gt
                rem = rem - n_gt

            tvec = lax.bitcast_convert_type(pv, jnp.float32)
            wait_prev_store()

            def fin_fast():
                @plsc.parallel_loop(0, NV, unroll=16)
                def _(j):
                    v = row_v[pl.ds(j * 16, 16)]
                    out_v[pl.ds(j * 16, 16)] = jnp.where(v >= tvec, v, 0.0)

            def fin_slow():
                @plsc.parallel_loop(0, NV, unroll=8,
                                    carry=jnp.zeros((16,), jnp.int32))
                def _(j, eqc):
                    v = row_v[pl.ds(j * 16, 16)]
                    gt = v > tvec
                    eq = v == tvec
                    ei = eq.astype(jnp.int32)
                    rank = eqc + jnp.cumsum(ei) - 1
                    keep = gt | (eq & (rank < rem))
                    out_v[pl.ds(j * 16, 16)] = jnp.where(keep, v, 0.0)
                    return eqc + _popcnt(eq)

            lax.cond(jnp.max(eq_total) == jnp.max(rem), fin_fast, fin_slow)

        load(row_a, sem_la, 0)

        def pair_body(i, _):
            r = i * 2
            load(row_b, sem_lb, r + 1)
            load_wait(row_a, sem_la, r)
            process(row_a, out_a,
                    lambda: pl.when(i > 0)(lambda: store_wait(out_a, sem_sa,
                                                             r - 2)))
            store(out_a, sem_sa, r)
            @pl.when(i < rows_per_w // 2 - 1)
            def _():
                load(row_a, sem_la, r + 2)
            load_wait(row_b, sem_lb, r + 1)
            process(row_b, out_b,
                    lambda: pl.when(i > 0)(lambda: store_wait(out_b, sem_sb,
                                                              r - 1)))
            store(out_b, sem_sb, r + 1)
            return 0

        lax.fori_loop(0, rows_per_w // 2, pair_body, 0)
        store_wait(out_a, sem_sa, rows_per_w - 2)
        store_wait(out_b, sem_sb, rows_per_w - 1)

    mesh = plsc.VectorSubcoreMesh(core_axis_name="c", subcore_axis_name="s")
    f = pl.kernel(
        body,
        mesh=mesh,
        compiler_params=pltpu.CompilerParams(needs_layout_passes=False),
        out_type=jax.ShapeDtypeStruct((nrows * DICT_SIZE,), jnp.float32),
        scratch_types=[
            pltpu.VMEM((DICT_SIZE,), jnp.float32),
            pltpu.VMEM((DICT_SIZE,), jnp.float32),
            pltpu.VMEM((DICT_SIZE,), jnp.float32),
            pltpu.VMEM((DICT_SIZE,), jnp.float32),
            pltpu.VMEM((DICT_SIZE,), jnp.float32),
            pltpu.VMEM((512,), jnp.int32),
            pltpu.VMEM((512,), jnp.int32),
            pltpu.SemaphoreType.DMA,
            pltpu.SemaphoreType.DMA,
            pltpu.SemaphoreType.DMA,
            pltpu.SemaphoreType.DMA,
        ],
    )
    return f(post_flat)


def _encode_body(x_ref, w_ref, b_dec_ref, enc_b_ref, out_ref):
    a = x_ref[...] - b_dec_ref[...]
    acc = lax.dot_general(a, w_ref[...], (((1,), (1,)), ((), ())),
                          preferred_element_type=jnp.float32)
    out_ref[...] = jnp.maximum(acc + enc_b_ref[...], 0.0)


def _encode(x, enc_W, enc_b, b_dec, m_blk=1024, f_blk=1024):
    nb = x.shape[0]
    grid = (nb // m_blk, DICT_SIZE // f_blk)
    return pl.pallas_call(
        _encode_body,
        grid=grid,
        in_specs=[
            pl.BlockSpec((m_blk, ACT_DIM), lambda i, j: (i, 0)),
            pl.BlockSpec((f_blk, ACT_DIM), lambda i, j: (j, 0)),
            pl.BlockSpec((1, ACT_DIM), lambda i, j: (0, 0)),
            pl.BlockSpec((1, f_blk), lambda i, j: (0, j)),
        ],
        out_specs=pl.BlockSpec((m_blk, f_blk), lambda i, j: (i, j)),
        out_shape=jax.ShapeDtypeStruct((nb, DICT_SIZE), jnp.float32),
    )(x, enc_W, b_dec.reshape(1, ACT_DIM), enc_b.reshape(1, DICT_SIZE))


def _decode_body(e_ref, w_ref, b_ref, out_ref):
    k = pl.program_id(1)

    @pl.when(k == 0)
    def _init():
        out_ref[...] = jnp.broadcast_to(b_ref[...], out_ref.shape)

    out_ref[...] += lax.dot_general(
        e_ref[...], w_ref[...], (((1,), (1,)), ((), ())),
        preferred_element_type=jnp.float32)


def _decode(encoded, dec_W, b_dec, m_blk=1024, k_blk=1024):
    nb = encoded.shape[0]
    grid = (nb // m_blk, DICT_SIZE // k_blk)
    return pl.pallas_call(
        _decode_body,
        grid=grid,
        in_specs=[
            pl.BlockSpec((m_blk, k_blk), lambda i, k: (i, k)),
            pl.BlockSpec((ACT_DIM, k_blk), lambda i, k: (0, k)),
            pl.BlockSpec((1, ACT_DIM), lambda i, k: (0, 0)),
        ],
        out_specs=pl.BlockSpec((m_blk, ACT_DIM), lambda i, k: (i, 0)),
        out_shape=jax.ShapeDtypeStruct((nb, ACT_DIM), jnp.float32),
    )(encoded, dec_W, b_dec.reshape(1, ACT_DIM))


def kernel(x, enc_W, enc_b, dec_W, b_dec):
    nchunks = 8
    mb = B // nchunks
    outs = []
    for c in range(nchunks):
        xc = lax.slice_in_dim(x, c * mb, (c + 1) * mb)
        post = _encode(xc, enc_W, enc_b, b_dec)
        enc = _sc_topk_mask(post.reshape(-1), mb).reshape(mb, DICT_SIZE)
        outs.append(_decode(enc, dec_W, b_dec))
    return jnp.concatenate(outs, axis=0)


# speedup vs baseline: 10.8592x; 1.0003x over previous
import functools

import jax
import jax.numpy as jnp
from jax import lax
from jax.experimental import pallas as pl
from jax.experimental.pallas import tpu as pltpu

from jax.experimental.pallas import tpu_sc as plsc

ACT_DIM = 2048
DICT_SIZE = 16384
K = 64
B = 8192

NC = 2
NS = 16
NW = NC * NS
ROWS_PER_W = B // NW
NV = DICT_SIZE // 16


def _lane(vec, lane):
    return lax.gather(
        vec, jnp.full((16, 1), lane, jnp.int32),
        lax.GatherDimensionNumbers(offset_dims=(), collapsed_slice_dims=(0,),
                                   start_index_map=(0,)),
        (1,), mode=lax.GatherScatterMode.PROMISE_IN_BOUNDS)


def _popcnt(mask):
    return plsc.all_reduce_population_count(mask)


def _sc_topk_mask(post_flat, nrows=B):
    rows_per_w = nrows // NW
    def _suffix_scan(hist_ref, sufx_ref, nch):
        @plsc.parallel_loop(0, nch, unroll=4,
                            carry=jnp.zeros((16,), jnp.int32))
        def _(jj, carry):
            j = nch - 1 - jj
            v = hist_ref[pl.ds(j * 16, 16)]
            rv = lax.rev(v, (0,))
            c = jnp.cumsum(rv) + carry
            sufx_ref[pl.ds(j * 16, 16)] = lax.rev(c, (0,))
            return _lane(c, 15)

    def _find_bucket(sufx_ref, nch, rem):
        @plsc.parallel_loop(0, nch, unroll=4,
                            carry=jnp.zeros((16,), jnp.int32))
        def cnt(j, cnt):
            s = sufx_ref[pl.ds(j * 16, 16)]
            return cnt + _popcnt(s >= rem)
        bstar = cnt - 1
        nbkt = nch * 16
        nxt = jnp.minimum(bstar + 1, nbkt - 1)
        sv = plsc.load_gather(sufx_ref, [nxt])
        n_gt = jnp.where(bstar + 1 >= nbkt, 0, sv)
        m = plsc.load_gather(sufx_ref, [jnp.maximum(bstar, 0)])
        return bstar, n_gt, m

    def body(post_hbm, out_hbm, row_a, row_b, out_a, out_b, cand_v, hist_v,
             sufx_v, sem_la, sem_lb, sem_sa, sem_sb):
        iota16 = jnp.arange(16, dtype=jnp.int32)
        ones16 = jnp.ones((16,), jnp.int32)
        zeros16 = jnp.zeros((16,), jnp.int32)
        wid = lax.axis_index("s") * NC + lax.axis_index("c")
        row0 = wid * rows_per_w

        def load(buf, sem, r):
            pltpu.make_async_copy(
                post_hbm.at[pl.ds((row0 + r) * DICT_SIZE, DICT_SIZE)],
                buf, sem).start()

        def load_wait(buf, sem, r):
            pltpu.make_async_copy(
                post_hbm.at[pl.ds((row0 + r) * DICT_SIZE, DICT_SIZE)],
                buf, sem).wait()

        def store(buf, sem, r):
            pltpu.make_async_copy(
                buf, out_hbm.at[pl.ds((row0 + r) * DICT_SIZE, DICT_SIZE)],
                sem).start()

        def store_wait(buf, sem, r):
            pltpu.make_async_copy(
                buf, out_hbm.at[pl.ds((row0 + r) * DICT_SIZE, DICT_SIZE)],
                sem).wait()

        def process(row_v, out_v, wait_prev_store):

            def zero_hist(nch):
                @plsc.parallel_loop(0, nch, unroll=4)
                def _(j):
                    hist_v[pl.ds(j * 16, 16)] = zeros16
            zero_hist(32)

            @plsc.parallel_loop(0, NV, unroll=16)
            def _(j):
                v = row_v[pl.ds(j * 16, 16)]
                b = jnp.right_shift(lax.bitcast_convert_type(v, jnp.int32), 22)
                plsc.addupdate_scatter(hist_v, [b], ones16)

            _suffix_scan(hist_v, sufx_v, 32)
            b1, n_gt, m = _find_bucket(sufx_v, 32, jnp.full((16,), K, jnp.int32))
            rem = K - n_gt

            @plsc.parallel_loop(0, NV, unroll=8,
                                carry=jnp.zeros((16,), jnp.int32))
            def _(j, off):
                v = row_v[pl.ds(j * 16, 16)]
                b = jnp.right_shift(lax.bitcast_convert_type(v, jnp.int32), 22)
                mask = b >= b1
                mi = mask.astype(jnp.int32)
                pos = jnp.maximum(off + jnp.cumsum(mi) - 1, 0)
                plsc.store_scatter(cand_v, [pos], v, mask=mask)
                return off + _popcnt(mask)

            m_s = jnp.max(m)
            nloops = (m_s + 15) // 16

            pv = b1
            eq_total = m - n_gt
            for lo_prev, lo, fm, nch in ((22, 14, 0xFF, 16),
                                         (14, 6, 0xFF, 16),
                                         (6, 0, 0x3F, 4)):
                zero_hist(nch)

                def hl(j, _, lo_prev=lo_prev, lo=lo, fm=fm, pv=pv):
                    v = cand_v[pl.ds(j * 16, 16)]
                    bits = lax.bitcast_convert_type(v, jnp.int32)
                    valid = (j * 16 + iota16) < m
                    act = valid & (jnp.right_shift(bits, lo_prev) == pv)
                    field = jnp.right_shift(bits, lo) & fm
                    plsc.addupdate_scatter(hist_v, [field], ones16,
                                           mask=act)
                    return 0
                lax.fori_loop(0, nloops, hl, 0)

                _suffix_scan(hist_v, sufx_v, nch)
                bl, n_gt, m2 = _find_bucket(sufx_v, nch, rem)
                pv = jnp.left_shift(pv, lo_prev - lo) | bl
                eq_total = m2 - n_gt
                rem = rem - n_gt

            tvec = lax.bitcast_convert_type(pv, jnp.float32)
            wait_prev_store()

            def fin_fast():
                @plsc.parallel_loop(0, NV, unroll=16)
                def _(j):
                    v = row_v[pl.ds(j * 16, 16)]
                    out_v[pl.ds(j * 16, 16)] = jnp.where(v >= tvec, v, 0.0)

            def fin_slow():
                @plsc.parallel_loop(0, NV, unroll=8,
                                    carry=jnp.zeros((16,), jnp.int32))
                def _(j, eqc):
                    v = row_v[pl.ds(j * 16, 16)]
                    gt = v > tvec
                    eq = v == tvec
                    ei = eq.astype(jnp.int32)
                    rank = eqc + jnp.cumsum(ei) - 1
                    keep = gt | (eq & (rank < rem))
                    out_v[pl.ds(j * 16, 16)] = jnp.where(keep, v, 0.0)
                    return eqc + _popcnt(eq)

            lax.cond(jnp.max(eq_total) == jnp.max(rem), fin_fast, fin_slow)

        load(row_a, sem_la, 0)

        def pair_body(i, _):
            r = i * 2
            load(row_b, sem_lb, r + 1)
            load_wait(row_a, sem_la, r)
            process(row_a, out_a,
                    lambda: pl.when(i > 0)(lambda: store_wait(out_a, sem_sa,
                                                             r - 2)))
            store(out_a, sem_sa, r)
            @pl.when(i < rows_per_w // 2 - 1)
            def _():
                load(row_a, sem_la, r + 2)
            load_wait(row_b, sem_lb, r + 1)
            process(row_b, out_b,
                    lambda: pl.when(i > 0)(lambda: store_wait(out_b, sem_sb,
                                                              r - 1)))
            store(out_b, sem_sb, r + 1)
            return 0

        lax.fori_loop(0, rows_per_w // 2, pair_body, 0)
        store_wait(out_a, sem_sa, rows_per_w - 2)
        store_wait(out_b, sem_sb, rows_per_w - 1)

    mesh = plsc.VectorSubcoreMesh(core_axis_name="c", subcore_axis_name="s")
    f = pl.kernel(
        body,
        mesh=mesh,
        compiler_params=pltpu.CompilerParams(needs_layout_passes=False),
        cost_estimate=pl.CostEstimate(
            flops=8 * nrows * DICT_SIZE,
            bytes_accessed=8 * nrows * DICT_SIZE,
            transcendentals=0),
        out_type=jax.ShapeDtypeStruct((nrows * DICT_SIZE,), jnp.float32),
        scratch_types=[
            pltpu.VMEM((DICT_SIZE,), jnp.float32),
            pltpu.VMEM((DICT_SIZE,), jnp.float32),
            pltpu.VMEM((DICT_SIZE,), jnp.float32),
            pltpu.VMEM((DICT_SIZE,), jnp.float32),
            pltpu.VMEM((DICT_SIZE,), jnp.float32),
            pltpu.VMEM((512,), jnp.int32),
            pltpu.VMEM((512,), jnp.int32),
            pltpu.SemaphoreType.DMA,
            pltpu.SemaphoreType.DMA,
            pltpu.SemaphoreType.DMA,
            pltpu.SemaphoreType.DMA,
        ],
    )
    return f(post_flat)


def _encode_body(x_ref, w_ref, b_dec_ref, enc_b_ref, out_ref):
    a = x_ref[...] - b_dec_ref[...]
    acc = lax.dot_general(a, w_ref[...], (((1,), (1,)), ((), ())),
                          preferred_element_type=jnp.float32)
    out_ref[...] = jnp.maximum(acc + enc_b_ref[...], 0.0)


def _encode(x, enc_W, enc_b, b_dec, m_blk=1024, f_blk=1024):
    nb = x.shape[0]
    grid = (nb // m_blk, DICT_SIZE // f_blk)
    return pl.pallas_call(
        _encode_body,
        grid=grid,
        in_specs=[
            pl.BlockSpec((m_blk, ACT_DIM), lambda i, j: (i, 0)),
            pl.BlockSpec((f_blk, ACT_DIM), lambda i, j: (j, 0)),
            pl.BlockSpec((1, ACT_DIM), lambda i, j: (0, 0)),
            pl.BlockSpec((1, f_blk), lambda i, j: (0, j)),
        ],
        out_specs=pl.BlockSpec((m_blk, f_blk), lambda i, j: (i, j)),
        out_shape=jax.ShapeDtypeStruct((nb, DICT_SIZE), jnp.float32),
    )(x, enc_W, b_dec.reshape(1, ACT_DIM), enc_b.reshape(1, DICT_SIZE))


def _decode_body(e_ref, w_ref, b_ref, out_ref):
    k = pl.program_id(1)

    @pl.when(k == 0)
    def _init():
        out_ref[...] = jnp.broadcast_to(b_ref[...], out_ref.shape)

    out_ref[...] += lax.dot_general(
        e_ref[...], w_ref[...], (((1,), (1,)), ((), ())),
        preferred_element_type=jnp.float32)


def _decode(encoded, dec_W, b_dec, m_blk=1024, k_blk=1024):
    nb = encoded.shape[0]
    grid = (nb // m_blk, DICT_SIZE // k_blk)
    return pl.pallas_call(
        _decode_body,
        grid=grid,
        in_specs=[
            pl.BlockSpec((m_blk, k_blk), lambda i, k: (i, k)),
            pl.BlockSpec((ACT_DIM, k_blk), lambda i, k: (0, k)),
            pl.BlockSpec((1, ACT_DIM), lambda i, k: (0, 0)),
        ],
        out_specs=pl.BlockSpec((m_blk, ACT_DIM), lambda i, k: (i, 0)),
        out_shape=jax.ShapeDtypeStruct((nb, ACT_DIM), jnp.float32),
    )(encoded, dec_W, b_dec.reshape(1, ACT_DIM))


def kernel(x, enc_W, enc_b, dec_W, b_dec):
    nchunks = 8
    mb = B // nchunks
    outs = []
    for c in range(nchunks):
        xc = lax.slice_in_dim(x, c * mb, (c + 1) * mb)
        post = _encode(xc, enc_W, enc_b, b_dec)
        enc = _sc_topk_mask(post.reshape(-1), mb).reshape(mb, DICT_SIZE)
        outs.append(_decode(enc, dec_W, b_dec))
    return jnp.concatenate(outs, axis=0)


# speedup vs baseline: 10.8660x; 1.0006x over previous
import functools

import jax
import jax.numpy as jnp
from jax import lax
from jax.experimental import pallas as pl
from jax.experimental.pallas import tpu as pltpu

from jax.experimental.pallas import tpu_sc as plsc

ACT_DIM = 2048
DICT_SIZE = 16384
K = 64
B = 8192

NC = 2
NS = 16
NW = NC * NS
ROWS_PER_W = B // NW
NV = DICT_SIZE // 16


def _lane(vec, lane):
    return lax.gather(
        vec, jnp.full((16, 1), lane, jnp.int32),
        lax.GatherDimensionNumbers(offset_dims=(), collapsed_slice_dims=(0,),
                                   start_index_map=(0,)),
        (1,), mode=lax.GatherScatterMode.PROMISE_IN_BOUNDS)


def _popcnt(mask):
    return plsc.all_reduce_population_count(mask)


def _sc_topk_mask(post_flat, nrows=B):
    rows_per_w = nrows // NW
    def _suffix_scan(hist_ref, sufx_ref, nch):
        @plsc.parallel_loop(0, nch, unroll=8,
                            carry=jnp.zeros((16,), jnp.int32))
        def _(jj, carry):
            j = nch - 1 - jj
            v = hist_ref[pl.ds(j * 16, 16)]
            rv = lax.rev(v, (0,))
            c = jnp.cumsum(rv) + carry
            sufx_ref[pl.ds(j * 16, 16)] = lax.rev(c, (0,))
            return _lane(c, 15)

    def _find_bucket(sufx_ref, nch, rem):
        @plsc.parallel_loop(0, nch, unroll=8,
                            carry=jnp.zeros((16,), jnp.int32))
        def cnt(j, cnt):
            s = sufx_ref[pl.ds(j * 16, 16)]
            return cnt + _popcnt(s >= rem)
        bstar = cnt - 1
        nbkt = nch * 16
        nxt = jnp.minimum(bstar + 1, nbkt - 1)
        sv = plsc.load_gather(sufx_ref, [nxt])
        n_gt = jnp.where(bstar + 1 >= nbkt, 0, sv)
        m = plsc.load_gather(sufx_ref, [jnp.maximum(bstar, 0)])
        return bstar, n_gt, m

    def body(post_hbm, out_hbm, row_a, row_b, out_a, out_b, cand_v, hist_v,
             sufx_v, sem_la, sem_lb, sem_sa, sem_sb):
        iota16 = jnp.arange(16, dtype=jnp.int32)
        ones16 = jnp.ones((16,), jnp.int32)
        zeros16 = jnp.zeros((16,), jnp.int32)
        wid = lax.axis_index("s") * NC + lax.axis_index("c")
        row0 = wid * rows_per_w

        def load(buf, sem, r):
            pltpu.make_async_copy(
                post_hbm.at[pl.ds((row0 + r) * DICT_SIZE, DICT_SIZE)],
                buf, sem).start()

        def load_wait(buf, sem, r):
            pltpu.make_async_copy(
                post_hbm.at[pl.ds((row0 + r) * DICT_SIZE, DICT_SIZE)],
                buf, sem).wait()

        def store(buf, sem, r):
            pltpu.make_async_copy(
                buf, out_hbm.at[pl.ds((row0 + r) * DICT_SIZE, DICT_SIZE)],
                sem).start()

        def store_wait(buf, sem, r):
            pltpu.make_async_copy(
                buf, out_hbm.at[pl.ds((row0 + r) * DICT_SIZE, DICT_SIZE)],
                sem).wait()

        def process(row_v, out_v, wait_prev_store):

            def zero_hist(nch):
                @plsc.parallel_loop(0, nch, unroll=8)
                def _(j):
                    hist_v[pl.ds(j * 16, 16)] = zeros16
            zero_hist(32)

            @plsc.parallel_loop(0, NV, unroll=16)
            def _(j):
                v = row_v[pl.ds(j * 16, 16)]
                b = jnp.right_shift(lax.bitcast_convert_type(v, jnp.int32), 22)
                plsc.addupdate_scatter(hist_v, [b], ones16)

            _suffix_scan(hist_v, sufx_v, 32)
            b1, n_gt, m = _find_bucket(sufx_v, 32, jnp.full((16,), K, jnp.int32))
            rem = K - n_gt

            @plsc.parallel_loop(0, NV, unroll=12,
                                carry=jnp.zeros((16,), jnp.int32))
            def _(j, off):
                v = row_v[pl.ds(j * 16, 16)]
                b = jnp.right_shift(lax.bitcast_convert_type(v, jnp.int32), 22)
                mask = b >= b1
                mi = mask.astype(jnp.int32)
                pos = jnp.maximum(off + jnp.cumsum(mi) - 1, 0)
                plsc.store_scatter(cand_v, [pos], v, mask=mask)
                return off + _popcnt(mask)

            m_s = jnp.max(m)
            nloops = (m_s + 15) // 16

            pv = b1
            eq_total = m - n_gt
            for lo_prev, lo, fm, nch in ((22, 14, 0xFF, 16),
                                         (14, 6, 0xFF, 16),
                                         (6, 0, 0x3F, 4)):
                zero_hist(nch)

                @plsc.parallel_loop(0, nloops, unroll=2)
                def _(j, lo_prev=lo_prev, lo=lo, fm=fm, pv=pv):
                    v = cand_v[pl.ds(j * 16, 16)]
                    bits = lax.bitcast_convert_type(v, jnp.int32)
                    valid = (j * 16 + iota16) < m
                    act = valid & (jnp.right_shift(bits, lo_prev) == pv)
                    field = jnp.right_shift(bits, lo) & fm
                    plsc.addupdate_scatter(hist_v, [field], ones16,
                                           mask=act)

                _suffix_scan(hist_v, sufx_v, nch)
                bl, n_gt, m2 = _find_bucket(sufx_v, nch, rem)
                pv = jnp.left_shift(pv, lo_prev - lo) | bl
                eq_total = m2 - n_gt
                rem = rem - n_gt

            tvec = lax.bitcast_convert_type(pv, jnp.float32)
            wait_prev_store()

            def fin_fast():
                @plsc.parallel_loop(0, NV, unroll=16)
                def _(j):
                    v = row_v[pl.ds(j * 16, 16)]
                    out_v[pl.ds(j * 16, 16)] = jnp.where(v >= tvec, v, 0.0)

            def fin_slow():
                @plsc.parallel_loop(0, NV, unroll=8,
                                    carry=jnp.zeros((16,), jnp.int32))
                def _(j, eqc):
                    v = row_v[pl.ds(j * 16, 16)]
                    gt = v > tvec
                    eq = v == tvec
                    ei = eq.astype(jnp.int32)
                    rank = eqc + jnp.cumsum(ei) - 1
                    keep = gt | (eq & (rank < rem))
                    out_v[pl.ds(j * 16, 16)] = jnp.where(keep, v, 0.0)
                    return eqc + _popcnt(eq)

            lax.cond(jnp.max(eq_total) == jnp.max(rem), fin_fast, fin_slow)

        load(row_a, sem_la, 0)

        def pair_body(i, _):
            r = i * 2
            load(row_b, sem_lb, r + 1)
            load_wait(row_a, sem_la, r)
            process(row_a, out_a,
                    lambda: pl.when(i > 0)(lambda: store_wait(out_a, sem_sa,
                                                             r - 2)))
            store(out_a, sem_sa, r)
            @pl.when(i < rows_per_w // 2 - 1)
            def _():
                load(row_a, sem_la, r + 2)
            load_wait(row_b, sem_lb, r + 1)
            process(row_b, out_b,
                    lambda: pl.when(i > 0)(lambda: store_wait(out_b, sem_sb,
                                                              r - 1)))
            store(out_b, sem_sb, r + 1)
            return 0

        lax.fori_loop(0, rows_per_w // 2, pair_body, 0)
        store_wait(out_a, sem_sa, rows_per_w - 2)
        store_wait(out_b, sem_sb, rows_per_w - 1)

    mesh = plsc.VectorSubcoreMesh(core_axis_name="c", subcore_axis_name="s")
    f = pl.kernel(
        body,
        mesh=mesh,
        compiler_params=pltpu.CompilerParams(needs_layout_passes=False),
        cost_estimate=pl.CostEstimate(
            flops=8 * nrows * DICT_SIZE,
            bytes_accessed=8 * nrows * DICT_SIZE,
            transcendentals=0),
        out_type=jax.ShapeDtypeStruct((nrows * DICT_SIZE,), jnp.float32),
        scratch_types=[
            pltpu.VMEM((DICT_SIZE,), jnp.float32),
            pltpu.VMEM((DICT_SIZE,), jnp.float32),
            pltpu.VMEM((DICT_SIZE,), jnp.float32),
            pltpu.VMEM((DICT_SIZE,), jnp.float32),
            pltpu.VMEM((DICT_SIZE,), jnp.float32),
            pltpu.VMEM((512,), jnp.int32),
            pltpu.VMEM((512,), jnp.int32),
            pltpu.SemaphoreType.DMA,
            pltpu.SemaphoreType.DMA,
            pltpu.SemaphoreType.DMA,
            pltpu.SemaphoreType.DMA,
        ],
    )
    return f(post_flat)


def _encode_body(x_ref, w_ref, b_dec_ref, enc_b_ref, out_ref):
    a = x_ref[...] - b_dec_ref[...]
    acc = lax.dot_general(a, w_ref[...], (((1,), (1,)), ((), ())),
                          preferred_element_type=jnp.float32)
    out_ref[...] = jnp.maximum(acc + enc_b_ref[...], 0.0)


def _encode(x, enc_W, enc_b, b_dec, m_blk=1024, f_blk=1024):
    nb = x.shape[0]
    grid = (nb // m_blk, DICT_SIZE // f_blk)
    return pl.pallas_call(
        _encode_body,
        grid=grid,
        in_specs=[
            pl.BlockSpec((m_blk, ACT_DIM), lambda i, j: (i, 0)),
            pl.BlockSpec((f_blk, ACT_DIM), lambda i, j: (j, 0)),
            pl.BlockSpec((1, ACT_DIM), lambda i, j: (0, 0)),
            pl.BlockSpec((1, f_blk), lambda i, j: (0, j)),
        ],
        out_specs=pl.BlockSpec((m_blk, f_blk), lambda i, j: (i, j)),
        out_shape=jax.ShapeDtypeStruct((nb, DICT_SIZE), jnp.float32),
    )(x, enc_W, b_dec.reshape(1, ACT_DIM), enc_b.reshape(1, DICT_SIZE))


def _decode_body(e_ref, w_ref, b_ref, out_ref):
    k = pl.program_id(1)

    @pl.when(k == 0)
    def _init():
        out_ref[...] = jnp.broadcast_to(b_ref[...], out_ref.shape)

    out_ref[...] += lax.dot_general(
        e_ref[...], w_ref[...], (((1,), (1,)), ((), ())),
        preferred_element_type=jnp.float32)


def _decode(encoded, dec_W, b_dec, m_blk=1024, k_blk=1024):
    nb = encoded.shape[0]
    grid = (nb // m_blk, DICT_SIZE // k_blk)
    return pl.pallas_call(
        _decode_body,
        grid=grid,
        in_specs=[
            pl.BlockSpec((m_blk, k_blk), lambda i, k: (i, k)),
            pl.BlockSpec((ACT_DIM, k_blk), lambda i, k: (0, k)),
            pl.BlockSpec((1, ACT_DIM), lambda i, k: (0, 0)),
        ],
        out_specs=pl.BlockSpec((m_blk, ACT_DIM), lambda i, k: (i, 0)),
        out_shape=jax.ShapeDtypeStruct((nb, ACT_DIM), jnp.float32),
    )(encoded, dec_W, b_dec.reshape(1, ACT_DIM))


def kernel(x, enc_W, enc_b, dec_W, b_dec):
    nchunks = 8
    mb = B // nchunks
    outs = []
    for c in range(nchunks):
        xc = lax.slice_in_dim(x, c * mb, (c + 1) * mb)
        post = _encode(xc, enc_W, enc_b, b_dec)
        enc = _sc_topk_mask(post.reshape(-1), mb).reshape(mb, DICT_SIZE)
        outs.append(_decode(enc, dec_W, b_dec))
    return jnp.concatenate(outs, axis=0)


# speedup vs baseline: 10.8674x; 1.0001x over previous
import jax
import jax.numpy as jnp
from jax import lax
from jax.experimental import pallas as pl
from jax.experimental.pallas import tpu as pltpu

from jax.experimental.pallas import tpu_sc as plsc

ACT_DIM = 2048
DICT_SIZE = 16384
K = 64
B = 8192

NC = 2
NS = 16
NW = NC * NS
NV = DICT_SIZE // 16


def _lane(vec, lane):
    return lax.gather(
        vec, jnp.full((16, 1), lane, jnp.int32),
        lax.GatherDimensionNumbers(offset_dims=(), collapsed_slice_dims=(0,),
                                   start_index_map=(0,)),
        (1,), mode=lax.GatherScatterMode.PROMISE_IN_BOUNDS)


def _popcnt(mask):
    return plsc.all_reduce_population_count(mask)


def _sc_topk_mask(post_flat, nrows=B):
    rows_per_w = nrows // NW
    def _suffix_scan(hist_ref, sufx_ref, nch):
        @plsc.parallel_loop(0, nch, unroll=8,
                            carry=jnp.zeros((16,), jnp.int32))
        def _(jj, carry):
            j = nch - 1 - jj
            v = hist_ref[pl.ds(j * 16, 16)]
            rv = lax.rev(v, (0,))
            c = jnp.cumsum(rv) + carry
            sufx_ref[pl.ds(j * 16, 16)] = lax.rev(c, (0,))
            return _lane(c, 15)

    def _find_bucket(sufx_ref, nch, rem):
        @plsc.parallel_loop(0, nch, unroll=8,
                            carry=jnp.zeros((16,), jnp.int32))
        def cnt(j, cnt):
            s = sufx_ref[pl.ds(j * 16, 16)]
            return cnt + _popcnt(s >= rem)
        bstar = cnt - 1
        nbkt = nch * 16
        nxt = jnp.minimum(bstar + 1, nbkt - 1)
        sv = plsc.load_gather(sufx_ref, [nxt])
        n_gt = jnp.where(bstar + 1 >= nbkt, 0, sv)
        m = plsc.load_gather(sufx_ref, [jnp.maximum(bstar, 0)])
        return bstar, n_gt, m

    def body(post_hbm, out_hbm, row_a, row_b, out_a, out_b, cand_v, hist_v,
             sufx_v, sem_la, sem_lb, sem_sa, sem_sb):
        iota16 = jnp.arange(16, dtype=jnp.int32)
        ones16 = jnp.ones((16,), jnp.int32)
        zeros16 = jnp.zeros((16,), jnp.int32)
        wid = lax.axis_index("s") * NC + lax.axis_index("c")
        row0 = wid * rows_per_w

        def load(buf, sem, r):
            pltpu.make_async_copy(
                post_hbm.at[pl.ds((row0 + r) * DICT_SIZE, DICT_SIZE)],
                buf, sem).start()

        def load_wait(buf, sem, r):
            pltpu.make_async_copy(
                post_hbm.at[pl.ds((row0 + r) * DICT_SIZE, DICT_SIZE)],
                buf, sem).wait()

        def store(buf, sem, r):
            pltpu.make_async_copy(
                buf, out_hbm.at[pl.ds((row0 + r) * DICT_SIZE, DICT_SIZE)],
                sem).start()

        def store_wait(buf, sem, r):
            pltpu.make_async_copy(
                buf, out_hbm.at[pl.ds((row0 + r) * DICT_SIZE, DICT_SIZE)],
                sem).wait()

        def process(row_v, out_v, wait_prev_store):

            def zero_hist(nch):
                @plsc.parallel_loop(0, nch, unroll=8)
                def _(j):
                    hist_v[pl.ds(j * 16, 16)] = zeros16
            zero_hist(32)

            @plsc.parallel_loop(0, NV, unroll=16)
            def _(j):
                v = row_v[pl.ds(j * 16, 16)]
                b = jnp.right_shift(lax.bitcast_convert_type(v, jnp.int32), 22)
                plsc.addupdate_scatter(hist_v, [b], ones16)

            _suffix_scan(hist_v, sufx_v, 32)
            b1, n_gt, m = _find_bucket(sufx_v, 32, jnp.full((16,), K, jnp.int32))
            rem = K - n_gt

            @plsc.parallel_loop(0, NV, unroll=12,
                                carry=jnp.zeros((16,), jnp.int32))
            def _(j, off):
                v = row_v[pl.ds(j * 16, 16)]
                b = jnp.right_shift(lax.bitcast_convert_type(v, jnp.int32), 22)
                mask = b >= b1
                mi = mask.astype(jnp.int32)
                pos = jnp.maximum(off + jnp.cumsum(mi) - 1, 0)
                plsc.store_scatter(cand_v, [pos], v, mask=mask)
                return off + _popcnt(mask)

            m_s = jnp.max(m)
            nloops = (m_s + 15) // 16

            pv = b1
            eq_total = m - n_gt
            for lo_prev, lo, fm, nch in ((22, 14, 0xFF, 16),
                                         (14, 6, 0xFF, 16),
                                         (6, 0, 0x3F, 4)):
                zero_hist(nch)

                @plsc.parallel_loop(0, nloops, unroll=2)
                def _(j, lo_prev=lo_prev, lo=lo, fm=fm, pv=pv):
                    v = cand_v[pl.ds(j * 16, 16)]
                    bits = lax.bitcast_convert_type(v, jnp.int32)
                    valid = (j * 16 + iota16) < m
                    act = valid & (jnp.right_shift(bits, lo_prev) == pv)
                    field = jnp.right_shift(bits, lo) & fm
                    plsc.addupdate_scatter(hist_v, [field], ones16,
                                           mask=act)

                _suffix_scan(hist_v, sufx_v, nch)
                bl, n_gt, m2 = _find_bucket(sufx_v, nch, rem)
                pv = jnp.left_shift(pv, lo_prev - lo) | bl
                eq_total = m2 - n_gt
                rem = rem - n_gt

            tvec = lax.bitcast_convert_type(pv, jnp.float32)
            wait_prev_store()

            def fin_fast():
                @plsc.parallel_loop(0, NV, unroll=16)
                def _(j):
                    v = row_v[pl.ds(j * 16, 16)]
                    out_v[pl.ds(j * 16, 16)] = jnp.where(v >= tvec, v, 0.0)

            def fin_slow():
                @plsc.parallel_loop(0, NV, unroll=8,
                                    carry=jnp.zeros((16,), jnp.int32))
                def _(j, eqc):
                    v = row_v[pl.ds(j * 16, 16)]
                    gt = v > tvec
                    eq = v == tvec
                    ei = eq.astype(jnp.int32)
                    rank = eqc + jnp.cumsum(ei) - 1
                    keep = gt | (eq & (rank < rem))
                    out_v[pl.ds(j * 16, 16)] = jnp.where(keep, v, 0.0)
                    return eqc + _popcnt(eq)

            lax.cond(jnp.max(eq_total) == jnp.max(rem), fin_fast, fin_slow)

        load(row_a, sem_la, 0)

        def pair_body(i, _):
            r = i * 2
            load(row_b, sem_lb, r + 1)
            load_wait(row_a, sem_la, r)
            process(row_a, out_a,
                    lambda: pl.when(i > 0)(lambda: store_wait(out_a, sem_sa,
                                                             r - 2)))
            store(out_a, sem_sa, r)
            @pl.when(i < rows_per_w // 2 - 1)
            def _():
                load(row_a, sem_la, r + 2)
            load_wait(row_b, sem_lb, r + 1)
            process(row_b, out_b,
                    lambda: pl.when(i > 0)(lambda: store_wait(out_b, sem_sb,
                                                              r - 1)))
            store(out_b, sem_sb, r + 1)
            return 0

        lax.fori_loop(0, rows_per_w // 2, pair_body, 0)
        store_wait(out_a, sem_sa, rows_per_w - 2)
        store_wait(out_b, sem_sb, rows_per_w - 1)

    mesh = plsc.VectorSubcoreMesh(core_axis_name="c", subcore_axis_name="s")
    f = pl.kernel(
        body,
        mesh=mesh,
        compiler_params=pltpu.CompilerParams(needs_layout_passes=False),
        cost_estimate=pl.CostEstimate(
            flops=8 * nrows * DICT_SIZE,
            bytes_accessed=8 * nrows * DICT_SIZE,
            transcendentals=0),
        out_type=jax.ShapeDtypeStruct((nrows * DICT_SIZE,), jnp.float32),
        scratch_types=[
            pltpu.VMEM((DICT_SIZE,), jnp.float32),
            pltpu.VMEM((DICT_SIZE,), jnp.float32),
            pltpu.VMEM((DICT_SIZE,), jnp.float32),
            pltpu.VMEM((DICT_SIZE,), jnp.float32),
            pltpu.VMEM((DICT_SIZE,), jnp.float32),
            pltpu.VMEM((512,), jnp.int32),
            pltpu.VMEM((512,), jnp.int32),
            pltpu.SemaphoreType.DMA,
            pltpu.SemaphoreType.DMA,
            pltpu.SemaphoreType.DMA,
            pltpu.SemaphoreType.DMA,
        ],
    )
    return f(post_flat)


def _encode_body(x_ref, w_ref, b_dec_ref, enc_b_ref, out_ref):
    a = x_ref[...] - b_dec_ref[...]
    acc = lax.dot_general(a, w_ref[...], (((1,), (1,)), ((), ())),
                          preferred_element_type=jnp.float32)
    out_ref[...] = jnp.maximum(acc + enc_b_ref[...], 0.0)


def _encode(x, enc_W, enc_b, b_dec, m_blk=1024, f_blk=1024):
    nb = x.shape[0]
    grid = (nb // m_blk, DICT_SIZE // f_blk)
    return pl.pallas_call(
        _encode_body,
        grid=grid,
        in_specs=[
            pl.BlockSpec((m_blk, ACT_DIM), lambda i, j: (i, 0)),
            pl.BlockSpec((f_blk, ACT_DIM), lambda i, j: (j, 0)),
            pl.BlockSpec((1, ACT_DIM), lambda i, j: (0, 0)),
            pl.BlockSpec((1, f_blk), lambda i, j: (0, j)),
        ],
        out_specs=pl.BlockSpec((m_blk, f_blk), lambda i, j: (i, j)),
        out_shape=jax.ShapeDtypeStruct((nb, DICT_SIZE), jnp.float32),
    )(x, enc_W, b_dec.reshape(1, ACT_DIM), enc_b.reshape(1, DICT_SIZE))


def _decode_body(e_ref, w_ref, b_ref, out_ref):
    k = pl.program_id(1)

    @pl.when(k == 0)
    def _init():
        out_ref[...] = jnp.broadcast_to(b_ref[...], out_ref.shape)

    out_ref[...] += lax.dot_general(
        e_ref[...], w_ref[...], (((1,), (1,)), ((), ())),
        preferred_element_type=jnp.float32)


def _decode(encoded, dec_W, b_dec, m_blk=1024, k_blk=1024):
    nb = encoded.shape[0]
    grid = (nb // m_blk, DICT_SIZE // k_blk)
    return pl.pallas_call(
        _decode_body,
        grid=grid,
        in_specs=[
            pl.BlockSpec((m_blk, k_blk), lambda i, k: (i, k)),
            pl.BlockSpec((ACT_DIM, k_blk), lambda i, k: (0, k)),
            pl.BlockSpec((1, ACT_DIM), lambda i, k: (0, 0)),
        ],
        out_specs=pl.BlockSpec((m_blk, ACT_DIM), lambda i, k: (i, 0)),
        out_shape=jax.ShapeDtypeStruct((nb, ACT_DIM), jnp.float32),
    )(encoded, dec_W, b_dec.reshape(1, ACT_DIM))


def kernel(x, enc_W, enc_b, dec_W, b_dec):
    nchunks = 8
    mb = B // nchunks
    outs = []
    for c in range(nchunks):
        xc = lax.slice_in_dim(x, c * mb, (c + 1) * mb)
        post = _encode(xc, enc_W, enc_b, b_dec)
        enc = _sc_topk_mask(post.reshape(-1), mb).reshape(mb, DICT_SIZE)
        outs.append(_decode(enc, dec_W, b_dec))
    return jnp.concatenate(outs, axis=0)
